# Initial kernel scaffold; baseline (speedup 1.0000x reference)
import jax
import jax.numpy as jnp
from jax.experimental import pallas as pl


def kernel(x, e_emb, c_emb, r_emb, W1, b1, W2, b2, Wcc1, bcc1, Wcc2, bcc2):
    raise NotImplementedError("write your pallas kernel here")


# speedup vs baseline: 1.7520x; 1.7520x over previous
import functools

import jax
import jax.numpy as jnp
from jax import lax
from jax.experimental import pallas as pl
from jax.experimental.pallas import tpu as pltpu
from jax.experimental.pallas import tpu_sc as plsc

NV = 1000
D = 128
NW = 32
PW = 832
MPAD = NW * PW
CH = 64
NCH = PW // CH
NG = [410] * 6 + [409] * 4

_IDXSPEC = []


def _row(g, col):
    _IDXSPEC.append((g, col))
    return len(_IDXSPEC) - 1


_TASKS = [
    ("relu", dict(lrow=0, ops=[(3, _row(0, 6)), (4, _row(0, 7))])),
    ("dot", dict(lrow=1, plus=[(0, _row(1, 6))], c=(1, _row(1, 7)))),
    ("abs", dict(lrow=2, plus=[(0, _row(2, 5)), (2, _row(2, 6))], minus=(0, _row(2, 7)))),
    ("abs", dict(lrow=3, plus=[(0, _row(3, 4)), (2, _row(3, 5)), (2, _row(3, 6))], minus=(0, _row(3, 7)))),
    ("abs", dict(lrow=4, plus=[(0, _row(4, 3)), (2, _row(4, 4)), (2, _row(4, 5)), (2, _row(4, 6))], minus=(0, _row(4, 7)))),
    ("q", dict(qblk=0, a=(0, _row(5, 3)), b=(2, _row(5, 4)))),
    ("q", dict(qblk=1, a=(0, _row(5, 5)), b=(2, _row(5, 6)))),
    ("t", dict(tblk=0, t=(0, _row(5, 7)))),
    ("q", dict(qblk=2, a=(0, _row(6, 1)), b=(2, _row(6, 2)))),
    ("q", dict(qblk=3, a=(0, _row(6, 3)), b=(2, _row(6, 4)))),
    ("q", dict(qblk=4, a=(0, _row(6, 5)), b=(2, _row(6, 6)))),
    ("t", dict(tblk=1, t=(0, _row(6, 7)))),
    ("dot", dict(lrow=5, plus=[(0, _row(7, 5)), (2, _row(7, 6))], c=(1, _row(7, 7)))),
    ("dot", dict(lrow=6, plus=[(0, _row(8, 4)), (2, _row(8, 5)), (2, _row(8, 6))], c=(1, _row(8, 7)))),
    ("dot", dict(lrow=7, plus=[(0, _row(9, 3)), (2, _row(9, 4)), (2, _row(9, 5)), (2, _row(9, 6))], c=(1, _row(9, 7)))),
]
_NIDX = len(_IDXSPEC)


def _build_idx(x):
    rows = []
    for g, col in _IDXSPEC:
        flat = x[g::10, :, col].reshape(-1)
        rows.append(jnp.pad(flat, (0, MPAD - flat.shape[0])))
    return jnp.concatenate(rows)


def _k1_body(c_ref, w_ref, b_ref, a_out, b_out):
    cc = c_ref[...]
    a_out[...] = jnp.dot(cc, w_ref[0:D, :], preferred_element_type=jnp.float32)
    b_out[...] = jnp.dot(cc, w_ref[D:2 * D, :], preferred_element_type=jnp.float32) + b_ref[...]


def _derive_tables(c_emb, Wcc1, bcc1):
    return pl.pallas_call(
        _k1_body,
        out_shape=[jax.ShapeDtypeStruct((NV, D), jnp.float32),
                   jax.ShapeDtypeStruct((NV, D), jnp.float32)],
    )(c_emb, Wcc1, bcc1.reshape(1, D))


def _sc_body(e_hbm, c_hbm, r_hbm, a_hbm, b2_hbm, w2_hbm, idx_hbm,
             l8_out, q_out, t_out,
             i0, i1, i2, i3, i4, b0, b1, b2, b3, b4, lbuf, w2v, sem):
    tables = [e_hbm, c_hbm, r_hbm, a_hbm, b2_hbm]
    iscr = [i0, i1, i2, i3, i4]
    bscr = [b0, b1, b2, b3, b4]
    wid = lax.axis_index("s") * 2 + lax.axis_index("c")
    base = wid * PW

    pltpu.sync_copy(w2_hbm, w2v)

    def load_idx(rows):
        for n, r in enumerate(rows):
            pltpu.sync_copy(idx_hbm.at[pl.ds(r * MPAD + base, PW)], iscr[n])

    def gather(tab_slot, iscr_n, bscr_n, c):
        return pltpu.async_copy(
            tables[tab_slot].at[iscr[iscr_n].at[pl.ds(c * CH, CH)]],
            bscr[bscr_n], sem)

    def reduce_task(kind, spec):
        if kind == "relu":
            ops = spec["ops"]
            nplus, has_minus, has_c = len(ops), False, False
        elif kind == "abs":
            ops = spec["plus"] + [spec["minus"]]
            nplus, has_minus, has_c = len(spec["plus"]), True, False
        else:
            ops = spec["plus"] + [spec["c"]]
            nplus, has_minus, has_c = len(spec["plus"]), False, True
        load_idx([r for _, r in ops])

        lane = lax.iota(jnp.int32, 16)
        w2k = [w2v[pl.ds(k * 16, 16)] for k in range(8)]

        def chunk(c, _):
            descs = [gather(t, n, n, c) for n, (t, _) in enumerate(ops)]
            for dsc in descs:
                dsc.wait()

            def vblock(vb, _):
                def item(j, vec):
                    i = vb * 16 + j
                    acc = None
                    for k in range(8):
                        sl = pl.ds(k * 16, 16)
                        s = bscr[0][i, sl]
                        for n in range(1, nplus):
                            s = s + bscr[n][i, sl]
                        if has_minus:
                            v = jnp.abs(s - bscr[nplus][i, sl])
                        elif has_c:
                            v = s * bscr[nplus][i, sl]
                        else:
                            v = jnp.maximum(s, 0.0) * w2k[k]
                        acc = v if acc is None else acc + v
                    tot = jnp.sum(acc)
                    return jnp.where(lane == j, tot, vec)

                vec = lax.fori_loop(0, 16, item, jnp.zeros((16,), jnp.float32))
                lbuf[pl.ds(c * CH + vb * 16, 16)] = -vec if kind == "abs" else vec
                return 0

            lax.fori_loop(0, CH // 16, vblock, 0)
            return 0

        lax.fori_loop(0, NCH, chunk, 0)
        pltpu.sync_copy(lbuf, l8_out.at[pl.ds(spec["lrow"] * MPAD + base, PW)])

    def q_task(spec):
        load_idx([spec["a"][1], spec["b"][1]])
        off = spec["qblk"] * MPAD + base

        def chunk(c, _):
            d0 = gather(spec["a"][0], 0, 0, c)
            d1 = gather(spec["b"][0], 1, 1, c)
            d0.wait()
            d1.wait()

            def item(i, _):
                for k in range(8):
                    sl = pl.ds(k * 16, 16)
                    b2[i, sl] = b0[i, sl] + b1[i, sl]
                return 0

            lax.fori_loop(0, CH, item, 0)
            pltpu.sync_copy(b2, q_out.at[pl.ds(off + c * CH, CH)])
            return 0

        lax.fori_loop(0, NCH, chunk, 0)

    def t_task(spec):
        load_idx([spec["t"][1]])
        off = spec["tblk"] * MPAD + base

        def chunk(c, _):
            gather(spec["t"][0], 0, 0, c).wait()
            pltpu.sync_copy(b0, t_out.at[pl.ds(off + c * CH, CH)])
            return 0

        lax.fori_loop(0, NCH, chunk, 0)

    for kind, spec in _TASKS:
        if kind == "q":
            q_task(spec)
        elif kind == "t":
            t_task(spec)
        else:
            reduce_task(kind, spec)


def _sc_call(e_emb, c_emb, r_emb, A, B2h, w2, idx):
    mesh = plsc.VectorSubcoreMesh(core_axis_name="c", subcore_axis_name="s")
    fn = functools.partial(
        pl.kernel,
        mesh=mesh,
        compiler_params=pltpu.CompilerParams(needs_layout_passes=False),
        out_type=[
            jax.ShapeDtypeStruct((8 * MPAD,), jnp.float32),
            jax.ShapeDtypeStruct((5 * MPAD, D), jnp.float32),
            jax.ShapeDtypeStruct((2 * MPAD, D), jnp.float32),
        ],
        scratch_types=(
            [pltpu.VMEM((PW,), jnp.int32) for _ in range(5)]
            + [pltpu.VMEM((CH, D), jnp.float32) for _ in range(5)]
            + [pltpu.VMEM((PW,), jnp.float32),
               pltpu.VMEM((D,), jnp.float32),
               pltpu.SemaphoreType.DMA]
        ),
    )(_sc_body)
    return fn(e_emb, c_emb, r_emb, A, B2h, w2, idx)


_BM = 1024
_NB = MPAD // _BM


def _att_body(nq, q_refs, t_ref, w1_ref, b1_ref, w2_ref, out_ref):
    w1 = w1_ref[...]
    w2 = w2_ref[...]
    b1 = b1_ref[...]
    qs = [q[...] for q in q_refs]
    zs = [jnp.dot(jnp.maximum(jnp.dot(q, w1, preferred_element_type=jnp.float32) + b1, 0.0),
                  w2, preferred_element_type=jnp.float32) for q in qs]
    m = zs[0]
    for z in zs[1:]:
        m = jnp.maximum(m, z)
    es = [jnp.exp(z - m) for z in zs]
    den = es[0]
    for e in es[1:]:
        den = den + e
    qe = es[0] * qs[0]
    for e, q in zip(es[1:], qs[1:]):
        qe = qe + e * q
    qe = qe / den
    out_ref[...] = -jnp.sum(jnp.abs(qe - t_ref[...]), axis=1)


def _att_call(nq, qblks, tblk, qcat, tcat, W1, b1, W2):
    def body(*refs):
        _att_body(nq, refs[0:nq], refs[nq], refs[nq + 1], refs[nq + 2], refs[nq + 3], refs[nq + 4])

    qspecs = [pl.BlockSpec((_BM, D), lambda i, j=jb: (j * _NB + i, 0)) for jb in qblks]
    return pl.pallas_call(
        body,
        grid=(_NB,),
        in_specs=qspecs + [
            pl.BlockSpec((_BM, D), lambda i, j=tblk: (j * _NB + i, 0)),
            pl.BlockSpec((D, D), lambda i: (0, 0)),
            pl.BlockSpec((1, D), lambda i: (0, 0)),
            pl.BlockSpec((D, D), lambda i: (0, 0)),
        ],
        out_specs=pl.BlockSpec((_BM,), lambda i: (i,)),
        out_shape=jax.ShapeDtypeStruct((MPAD,), jnp.float32),
    )(*([qcat] * nq), tcat, W1, b1.reshape(1, D), W2)


def _loss_body(lg_ref, out_ref):
    acc = jnp.zeros((1, D), jnp.float32)
    lane = lax.broadcasted_iota(jnp.int32, (1, D), 1)
    rows = lax.broadcasted_iota(jnp.int32, (MPAD // 64, 63), 0)
    for g in range(10):
        lg = lg_ref[g]
        z = lg[:, 0:1] - lg[:, 1:]
        ls = jnp.minimum(z, 0.0) - jnp.log1p(jnp.exp(-jnp.abs(z)))
        s = jnp.sum(jnp.where(rows < NG[g], ls, 0.0))
        acc = acc + jnp.where(lane == g, -s / (NG[g] * 63), 0.0)
    out_ref[...] = acc


def _loss_call(LG):
    out = pl.pallas_call(
        _loss_body,
        out_shape=jax.ShapeDtypeStruct((1, D), jnp.float32),
    )(LG)
    return out[0, :10]


def kernel(x, e_emb, c_emb, r_emb, W1, b1, W2, b2, Wcc1, bcc1, Wcc2, bcc2):
    A, B2h = _derive_tables(c_emb, Wcc1, bcc1)
    idx = _build_idx(x)
    l8, qcat, tcat = _sc_call(e_emb, c_emb, r_emb, A, B2h, Wcc2.reshape(D), idx)
    l8 = l8.reshape(8, MPAD)
    l5 = _att_call(2, [0, 1], 0, qcat, tcat, W1, b1, W2)
    l6 = _att_call(3, [2, 3, 4], 1, qcat, tcat, W1, b1, W2)
    LG = jnp.stack([l8[0], l8[1], l8[2], l8[3], l8[4], l5, l6,
                    l8[5], l8[6], l8[7]]).reshape(10, MPAD // 64, 64)
    return _loss_call(LG)


# speedup vs baseline: 3.5612x; 2.0327x over previous
import functools

import jax
import jax.numpy as jnp
from jax import lax
from jax.experimental import pallas as pl
from jax.experimental.pallas import tpu as pltpu
from jax.experimental.pallas import tpu_sc as plsc

NV = 1000
D = 128
NW = 32
PW = 832
MPAD = NW * PW
CH = 64
NCH = PW // CH
NG = [410] * 6 + [409] * 4

_TASKS = [
    ("relu", dict(g=0, lrow=0, ops=[(3, 6), (4, 7)])),
    ("dot", dict(g=1, lrow=1, plus=[(0, 6)], c=(1, 7))),
    ("abs", dict(g=2, lrow=2, plus=[(0, 5), (2, 6)], minus=(0, 7))),
    ("abs", dict(g=3, lrow=3, plus=[(0, 4), (2, 5), (2, 6)], minus=(0, 7))),
    ("abs", dict(g=4, lrow=4, plus=[(0, 3), (2, 4), (2, 5), (2, 6)], minus=(0, 7))),
    ("q", dict(g=5, qblk=0, a=(0, 3), b=(2, 4))),
    ("q", dict(g=5, qblk=1, a=(0, 5), b=(2, 6))),
    ("t", dict(g=5, tblk=0, t=(0, 7))),
    ("q", dict(g=6, qblk=2, a=(0, 1), b=(2, 2))),
    ("q", dict(g=6, qblk=3, a=(0, 3), b=(2, 4))),
    ("q", dict(g=6, qblk=4, a=(0, 5), b=(2, 6))),
    ("t", dict(g=6, tblk=1, t=(0, 7))),
    ("dot", dict(g=7, lrow=5, plus=[(0, 5), (2, 6)], c=(1, 7))),
    ("dot", dict(g=8, lrow=6, plus=[(0, 4), (2, 5), (2, 6)], c=(1, 7))),
    ("dot", dict(g=9, lrow=7, plus=[(0, 3), (2, 4), (2, 5), (2, 6)], c=(1, 7))),
]


def _k1_body(c_ref, w_ref, b_ref, a_out, b_out):
    cc = c_ref[...]
    a_out[...] = jnp.dot(cc, w_ref[0:D, :], preferred_element_type=jnp.float32)
    b_out[...] = jnp.dot(cc, w_ref[D:2 * D, :], preferred_element_type=jnp.float32) + b_ref[...]


def _derive_tables(c_emb, Wcc1, bcc1):
    return pl.pallas_call(
        _k1_body,
        out_shape=[jax.ShapeDtypeStruct((NV, D), jnp.float32),
                   jax.ShapeDtypeStruct((NV, D), jnp.float32)],
    )(c_emb, Wcc1, bcc1.reshape(1, D))


def _sc_body(e_hbm, c_hbm, r_hbm, a_hbm, b2_hbm, w2_hbm, xf_hbm,
             l8_out, q_out, t_out,
             i0, i1, i2, i3, i4, i5, i6, b0, b1, b2, b3, b4,
             xrow, lbuf, w2v, sem):
    tables = [e_hbm, c_hbm, r_hbm, a_hbm, b2_hbm]
    iscr = [i0, i1, i2, i3, i4, i5, i6]
    bscr = [b0, b1, b2, b3, b4]
    wid = lax.axis_index("s") * 2 + lax.axis_index("c")
    base = wid * PW
    lane = lax.iota(jnp.int32, 16)

    pltpu.sync_copy(w2_hbm, w2v)

    def stage_x(g):
        descs = []
        for j in range(13):
            n_eff = jnp.minimum(wid * 13 + j, NG[g] - 1)
            descs.append(pltpu.async_copy(
                xf_hbm.at[pl.ds((g + 10 * n_eff) * 512, 512)],
                xrow.at[pl.ds(j * 512, 512)], sem))
        for d in descs:
            d.wait()

    def extract_cols(cols):
        def ext(vb, _):
            p = vb * 16 + lane
            fb = (p >> 6) * 512 + (p & 63) * 8
            for col in cols:
                v = plsc.load_gather(xrow, [fb + col])
                iscr[col - 1][pl.ds(vb * 16, 16)] = v
            return 0

        lax.fori_loop(0, PW // 16, ext, 0)

    def gather(tab_slot, col, bscr_n, c):
        return pltpu.async_copy(
            tables[tab_slot].at[iscr[col - 1].at[pl.ds(c * CH, CH)]],
            bscr[bscr_n], sem)

    def reduce_task(kind, spec):
        if kind == "relu":
            ops = spec["ops"]
            nplus, has_minus, has_c = len(ops), False, False
        elif kind == "abs":
            ops = spec["plus"] + [spec["minus"]]
            nplus, has_minus, has_c = len(spec["plus"]), True, False
        else:
            ops = spec["plus"] + [spec["c"]]
            nplus, has_minus, has_c = len(spec["plus"]), False, True
        w2k = [w2v[pl.ds(k * 16, 16)] for k in range(8)]

        def chunk(c, _):
            descs = [gather(t, col, n, c) for n, (t, col) in enumerate(ops)]
            for dsc in descs:
                dsc.wait()

            def vblock(vb, _):
                def item(j, vec):
                    i = vb * 16 + j
                    acc = None
                    for k in range(8):
                        sl = pl.ds(k * 16, 16)
                        s = bscr[0][i, sl]
                        for n in range(1, nplus):
                            s = s + bscr[n][i, sl]
                        if has_minus:
                            v = jnp.abs(s - bscr[nplus][i, sl])
                        elif has_c:
                            v = s * bscr[nplus][i, sl]
                        else:
                            v = jnp.maximum(s, 0.0) * w2k[k]
                        acc = v if acc is None else acc + v
                    tot = jnp.sum(acc)
                    return jnp.where(lane == j, tot, vec)

                vec = lax.fori_loop(0, 16, item, jnp.zeros((16,), jnp.float32))
                lbuf[pl.ds(c * CH + vb * 16, 16)] = -vec if kind == "abs" else vec
                return 0

            lax.fori_loop(0, CH // 16, vblock, 0)
            return 0

        lax.fori_loop(0, NCH, chunk, 0)
        pltpu.sync_copy(lbuf, l8_out.at[pl.ds(spec["lrow"] * MPAD + base, PW)])

    def q_task(spec):
        off = spec["qblk"] * MPAD + base

        def chunk(c, _):
            d0 = gather(spec["a"][0], spec["a"][1], 0, c)
            d1 = gather(spec["b"][0], spec["b"][1], 1, c)
            d0.wait()
            d1.wait()

            def item(i, _):
                for k in range(8):
                    sl = pl.ds(k * 16, 16)
                    b2[i, sl] = b0[i, sl] + b1[i, sl]
                return 0

            lax.fori_loop(0, CH, item, 0)
            pltpu.sync_copy(b2, q_out.at[pl.ds(off + c * CH, CH)])
            return 0

        lax.fori_loop(0, NCH, chunk, 0)

    def t_task(spec):
        off = spec["tblk"] * MPAD + base

        def chunk(c, _):
            gather(spec["t"][0], spec["t"][1], 0, c).wait()
            pltpu.sync_copy(b0, t_out.at[pl.ds(off + c * CH, CH)])
            return 0

        lax.fori_loop(0, NCH, chunk, 0)

    def task_cols(kind, spec):
        if kind == "relu":
            return [c for _, c in spec["ops"]]
        if kind == "abs":
            return [c for _, c in spec["plus"]] + [spec["minus"][1]]
        if kind == "dot":
            return [c for _, c in spec["plus"]] + [spec["c"][1]]
        if kind == "q":
            return [spec["a"][1], spec["b"][1]]
        return [spec["t"][1]]

    for g in range(10):
        gtasks = [(k, s) for k, s in _TASKS if s["g"] == g]
        stage_x(g)
        cols = sorted({c for k, s in gtasks for c in task_cols(k, s)})
        extract_cols(cols)
        for kind, spec in gtasks:
            if kind == "q":
                q_task(spec)
            elif kind == "t":
                t_task(spec)
            else:
                reduce_task(kind, spec)


def _sc_call(e_emb, c_emb, r_emb, A, B2h, w2, xf):
    mesh = plsc.VectorSubcoreMesh(core_axis_name="c", subcore_axis_name="s")
    fn = functools.partial(
        pl.kernel,
        mesh=mesh,
        compiler_params=pltpu.CompilerParams(needs_layout_passes=False),
        out_type=[
            jax.ShapeDtypeStruct((8 * MPAD,), jnp.float32),
            jax.ShapeDtypeStruct((5 * MPAD, D), jnp.float32),
            jax.ShapeDtypeStruct((2 * MPAD, D), jnp.float32),
        ],
        scratch_types=(
            [pltpu.VMEM((PW,), jnp.int32) for _ in range(7)]
            + [pltpu.VMEM((CH, D), jnp.float32) for _ in range(5)]
            + [pltpu.VMEM((13 * 512,), jnp.int32),
               pltpu.VMEM((PW,), jnp.float32),
               pltpu.VMEM((D,), jnp.float32),
               pltpu.SemaphoreType.DMA]
        ),
    )(_sc_body)
    return fn(e_emb, c_emb, r_emb, A, B2h, w2, xf)


_BM = 1024
_NB = MPAD // _BM


def _att_body(nq, q_refs, t_ref, w1_ref, b1_ref, w2_ref, out_ref):
    w1 = w1_ref[...]
    w2 = w2_ref[...]
    b1 = b1_ref[...]
    qs = [q[...] for q in q_refs]
    zs = [jnp.dot(jnp.maximum(jnp.dot(q, w1, preferred_element_type=jnp.float32) + b1, 0.0),
                  w2, preferred_element_type=jnp.float32) for q in qs]
    m = zs[0]
    for z in zs[1:]:
        m = jnp.maximum(m, z)
    es = [jnp.exp(z - m) for z in zs]
    den = es[0]
    for e in es[1:]:
        den = den + e
    qe = es[0] * qs[0]
    for e, q in zip(es[1:], qs[1:]):
        qe = qe + e * q
    qe = qe / den
    out_ref[...] = -jnp.sum(jnp.abs(qe - t_ref[...]), axis=1)


def _att_call(nq, qblks, tblk, qcat, tcat, W1, b1, W2):
    def body(*refs):
        _att_body(nq, refs[0:nq], refs[nq], refs[nq + 1], refs[nq + 2], refs[nq + 3], refs[nq + 4])

    qspecs = [pl.BlockSpec((_BM, D), lambda i, j=jb: (j * _NB + i, 0)) for jb in qblks]
    return pl.pallas_call(
        body,
        grid=(_NB,),
        in_specs=qspecs + [
            pl.BlockSpec((_BM, D), lambda i, j=tblk: (j * _NB + i, 0)),
            pl.BlockSpec((D, D), lambda i: (0, 0)),
            pl.BlockSpec((1, D), lambda i: (0, 0)),
            pl.BlockSpec((D, D), lambda i: (0, 0)),
        ],
        out_specs=pl.BlockSpec((_BM,), lambda i: (i,)),
        out_shape=jax.ShapeDtypeStruct((MPAD,), jnp.float32),
    )(*([qcat] * nq), tcat, W1, b1.reshape(1, D), W2)


def _loss_body(lg_ref, out_ref):
    acc = jnp.zeros((1, D), jnp.float32)
    lane = lax.broadcasted_iota(jnp.int32, (1, D), 1)
    rows = lax.broadcasted_iota(jnp.int32, (MPAD // 64, 63), 0)
    for g in range(10):
        lg = lg_ref[g]
        z = lg[:, 0:1] - lg[:, 1:]
        ls = jnp.minimum(z, 0.0) - jnp.log1p(jnp.exp(-jnp.abs(z)))
        s = jnp.sum(jnp.where(rows < NG[g], ls, 0.0))
        acc = acc + jnp.where(lane == g, -s / (NG[g] * 63), 0.0)
    out_ref[...] = acc


def _loss_call(LG):
    out = pl.pallas_call(
        _loss_body,
        out_shape=jax.ShapeDtypeStruct((1, D), jnp.float32),
    )(LG)
    return out[0, :10]


def kernel(x, e_emb, c_emb, r_emb, W1, b1, W2, b2, Wcc1, bcc1, Wcc2, bcc2):
    A, B2h = _derive_tables(c_emb, Wcc1, bcc1)
    l8, qcat, tcat = _sc_call(e_emb, c_emb, r_emb, A, B2h, Wcc2.reshape(D),
                              x.reshape(-1))
    l8 = l8.reshape(8, MPAD)
    l5 = _att_call(2, [0, 1], 0, qcat, tcat, W1, b1, W2)
    l6 = _att_call(3, [2, 3, 4], 1, qcat, tcat, W1, b1, W2)
    LG = jnp.stack([l8[0], l8[1], l8[2], l8[3], l8[4], l5, l6,
                    l8[5], l8[6], l8[7]]).reshape(10, MPAD // 64, 64)
    return _loss_call(LG)


# speedup vs baseline: 3.9156x; 1.0995x over previous
import functools

import jax
import jax.numpy as jnp
from jax import lax
from jax.experimental import pallas as pl
from jax.experimental.pallas import tpu as pltpu
from jax.experimental.pallas import tpu_sc as plsc

NV = 1000
D = 128
NW = 32
PW = 832
MPAD = NW * PW
CH = 64
NCH = PW // CH
NG = [410] * 6 + [409] * 4

_TASKS = [
    ("relu", dict(g=0, lrow=0, ops=[(3, 6), (4, 7)])),
    ("dot", dict(g=1, lrow=1, plus=[(0, 6)], c=(1, 7))),
    ("abs", dict(g=2, lrow=2, plus=[(0, 5), (2, 6)], minus=(0, 7))),
    ("abs", dict(g=3, lrow=3, plus=[(0, 4), (2, 5), (2, 6)], minus=(0, 7))),
    ("abs", dict(g=4, lrow=4, plus=[(0, 3), (2, 4), (2, 5), (2, 6)], minus=(0, 7))),
    ("q", dict(g=5, qblk=0, a=(0, 3), b=(2, 4))),
    ("q", dict(g=5, qblk=1, a=(0, 5), b=(2, 6))),
    ("t", dict(g=5, tblk=0, t=(0, 7))),
    ("q", dict(g=6, qblk=2, a=(0, 1), b=(2, 2))),
    ("q", dict(g=6, qblk=3, a=(0, 3), b=(2, 4))),
    ("q", dict(g=6, qblk=4, a=(0, 5), b=(2, 6))),
    ("t", dict(g=6, tblk=1, t=(0, 7))),
    ("dot", dict(g=7, lrow=5, plus=[(0, 5), (2, 6)], c=(1, 7))),
    ("dot", dict(g=8, lrow=6, plus=[(0, 4), (2, 5), (2, 6)], c=(1, 7))),
    ("dot", dict(g=9, lrow=7, plus=[(0, 3), (2, 4), (2, 5), (2, 6)], c=(1, 7))),
]


def _k1_body(c_ref, w_ref, b_ref, a_out, b_out):
    cc = c_ref[...]
    a_out[...] = jnp.dot(cc, w_ref[0:D, :], preferred_element_type=jnp.float32)
    b_out[...] = jnp.dot(cc, w_ref[D:2 * D, :], preferred_element_type=jnp.float32) + b_ref[...]


def _derive_tables(c_emb, Wcc1, bcc1):
    return pl.pallas_call(
        _k1_body,
        out_shape=[jax.ShapeDtypeStruct((NV, D), jnp.float32),
                   jax.ShapeDtypeStruct((NV, D), jnp.float32)],
    )(c_emb, Wcc1, bcc1.reshape(1, D))


def _sc_body(e_hbm, c_hbm, r_hbm, a_hbm, b2_hbm, w2_hbm, xf_hbm,
             l8_out, q_out, t_out,
             i0, i1, i2, i3, i4, i5, i6, b0, b1, b2, b3, b4, b5, b6, b7, b8, b9,
             xrow, lbuf, w2v, semA, semB):
    tables = [e_hbm, c_hbm, r_hbm, a_hbm, b2_hbm]
    iscr = [i0, i1, i2, i3, i4, i5, i6]
    bscr = [b0, b1, b2, b3, b4, b5, b6, b7, b8, b9]
    wid = lax.axis_index("s") * 2 + lax.axis_index("c")
    base = wid * PW
    lane = lax.iota(jnp.int32, 16)

    pltpu.sync_copy(w2_hbm, w2v)

    def stage_x(g):
        def issue(j, _):
            n_eff = jnp.minimum(wid * 13 + j, NG[g] - 1)
            pltpu.async_copy(
                xf_hbm.at[pl.ds((g + 10 * n_eff) * 512, 512)],
                xrow.at[pl.ds(j * 512, 512)], semA)
            return 0

        def drain(j, _):
            pltpu.make_async_copy(
                xf_hbm.at[pl.ds(0, 512)], xrow.at[pl.ds(0, 512)], semA).wait()
            return 0

        lax.fori_loop(0, 13, issue, 0)
        lax.fori_loop(0, 13, drain, 0)

    def extract_cols(cols):
        def ext(vb, _):
            p = vb * 16 + lane
            fb = (p >> 6) * 512 + (p & 63) * 8
            for col in cols:
                v = plsc.load_gather(xrow, [fb + col])
                iscr[col - 1][pl.ds(vb * 16, 16)] = v
            return 0

        lax.fori_loop(0, PW // 16, ext, 0, unroll=2)

    def gather(tab_slot, col, buf, c, sem):
        return pltpu.async_copy(
            tables[tab_slot].at[iscr[col - 1].at[pl.ds(c * CH, CH)]],
            buf, sem)

    def pipeline(issue, drain, compute):
        issue(jnp.int32(0), 0)

        def pair(i, _):
            cA = 2 * i
            issue(cA + 1, 1)
            drain(0)
            compute(cA, 0)
            issue(cA + 2, 0)
            drain(1)
            compute(cA + 1, 1)
            return 0

        lax.fori_loop(0, (NCH - 1) // 2, pair, 0)
        drain(0)
        compute(jnp.int32(NCH - 1), 0)

    def reduce_task(kind, spec):
        if kind == "relu":
            ops = spec["ops"]
            nplus, has_minus, has_c = len(ops), False, False
        elif kind == "abs":
            ops = spec["plus"] + [spec["minus"]]
            nplus, has_minus, has_c = len(spec["plus"]), True, False
        else:
            ops = spec["plus"] + [spec["c"]]
            nplus, has_minus, has_c = len(spec["plus"]), False, True
        w2k = [w2v[pl.ds(k * 16, 16)] for k in range(8)]
        sets = [bscr[0:len(ops)], bscr[5:5 + len(ops)]]
        sems = [semA, semB]

        def issue(c, s):
            for n, (t, col) in enumerate(ops):
                gather(t, col, sets[s][n], c, sems[s])

        def drain(s):
            for n, (t, col) in enumerate(ops):
                pltpu.make_async_copy(
                    tables[t].at[iscr[col - 1].at[pl.ds(0, CH)]],
                    sets[s][n], sems[s]).wait()

        def compute(c, s):
            bset = sets[s]

            def vblock(vb, _):
                def item(j, vec):
                    i = vb * 16 + j
                    acc = None
                    for k in range(8):
                        sl = pl.ds(k * 16, 16)
                        s_ = bset[0][i, sl]
                        for n in range(1, nplus):
                            s_ = s_ + bset[n][i, sl]
                        if has_minus:
                            v = jnp.abs(s_ - bset[nplus][i, sl])
                        elif has_c:
                            v = s_ * bset[nplus][i, sl]
                        else:
                            v = jnp.maximum(s_, 0.0) * w2k[k]
                        acc = v if acc is None else acc + v
                    tot = jnp.sum(acc)
                    return jnp.where(lane == j, tot, vec)

                vec = lax.fori_loop(0, 16, item, jnp.zeros((16,), jnp.float32),
                                    unroll=2)
                lbuf[pl.ds(c * CH + vb * 16, 16)] = -vec if kind == "abs" else vec
                return 0

            lax.fori_loop(0, CH // 16, vblock, 0)

        pipeline(issue, drain, compute)
        pltpu.sync_copy(lbuf, l8_out.at[pl.ds(spec["lrow"] * MPAD + base, PW)])

    def q_task(spec):
        off = spec["qblk"] * MPAD + base
        ta, ca = spec["a"]
        tb, cb = spec["b"]
        sets = [(b0, b1, b2), (b5, b6, b7)]
        sems = [semA, semB]

        def issue(c, s):
            gather(ta, ca, sets[s][0], c, sems[s])
            gather(tb, cb, sets[s][1], c, sems[s])

        def drain(s):
            for n, (t, col) in enumerate([spec["a"], spec["b"]]):
                pltpu.make_async_copy(
                    tables[t].at[iscr[col - 1].at[pl.ds(0, CH)]],
                    sets[s][n], sems[s]).wait()

        def compute(c, s):
            ba, bb, bq = sets[s]

            def item(i, _):
                for k in range(8):
                    sl = pl.ds(k * 16, 16)
                    bq[i, sl] = ba[i, sl] + bb[i, sl]
                return 0

            lax.fori_loop(0, CH, item, 0, unroll=2)
            pltpu.sync_copy(bq, q_out.at[pl.ds(off + c * CH, CH)])

        pipeline(issue, drain, compute)

    def t_task(spec):
        off = spec["tblk"] * MPAD + base
        tt, ct = spec["t"]
        sets = [b0, b5]
        sems = [semA, semB]

        def issue(c, s):
            gather(tt, ct, sets[s], c, sems[s])

        def drain(s):
            pltpu.make_async_copy(
                tables[tt].at[iscr[ct - 1].at[pl.ds(0, CH)]],
                sets[s], sems[s]).wait()

        def compute(c, s):
            pltpu.sync_copy(sets[s], t_out.at[pl.ds(off + c * CH, CH)])

        pipeline(issue, drain, compute)

    def task_cols(kind, spec):
        if kind == "relu":
            return [c for _, c in spec["ops"]]
        if kind == "abs":
            return [c for _, c in spec["plus"]] + [spec["minus"][1]]
        if kind == "dot":
            return [c for _, c in spec["plus"]] + [spec["c"][1]]
        if kind == "q":
            return [spec["a"][1], spec["b"][1]]
        return [spec["t"][1]]

    for g in range(10):
        gtasks = [(k, s) for k, s in _TASKS if s["g"] == g]
        stage_x(g)
        cols = sorted({c for k, s in gtasks for c in task_cols(k, s)})
        extract_cols(cols)
        for kind, spec in gtasks:
            if kind == "q":
                q_task(spec)
            elif kind == "t":
                t_task(spec)
            else:
                reduce_task(kind, spec)


def _sc_call(e_emb, c_emb, r_emb, A, B2h, w2, xf):
    mesh = plsc.VectorSubcoreMesh(core_axis_name="c", subcore_axis_name="s")
    fn = functools.partial(
        pl.kernel,
        mesh=mesh,
        compiler_params=pltpu.CompilerParams(needs_layout_passes=False),
        out_type=[
            jax.ShapeDtypeStruct((8 * MPAD,), jnp.float32),
            jax.ShapeDtypeStruct((5 * MPAD, D), jnp.float32),
            jax.ShapeDtypeStruct((2 * MPAD, D), jnp.float32),
        ],
        scratch_types=(
            [pltpu.VMEM((PW,), jnp.int32) for _ in range(7)]
            + [pltpu.VMEM((CH, D), jnp.float32) for _ in range(10)]
            + [pltpu.VMEM((13 * 512,), jnp.int32),
               pltpu.VMEM((PW,), jnp.float32),
               pltpu.VMEM((D,), jnp.float32),
               pltpu.SemaphoreType.DMA,
               pltpu.SemaphoreType.DMA]
        ),
    )(_sc_body)
    return fn(e_emb, c_emb, r_emb, A, B2h, w2, xf)


_BM = 1024
_NB = MPAD // _BM


def _att_body(nq, q_refs, t_ref, w1_ref, b1_ref, w2_ref, out_ref):
    w1 = w1_ref[...]
    w2 = w2_ref[...]
    b1 = b1_ref[...]
    qs = [q[...] for q in q_refs]
    zs = [jnp.dot(jnp.maximum(jnp.dot(q, w1, preferred_element_type=jnp.float32) + b1, 0.0),
                  w2, preferred_element_type=jnp.float32) for q in qs]
    m = zs[0]
    for z in zs[1:]:
        m = jnp.maximum(m, z)
    es = [jnp.exp(z - m) for z in zs]
    den = es[0]
    for e in es[1:]:
        den = den + e
    qe = es[0] * qs[0]
    for e, q in zip(es[1:], qs[1:]):
        qe = qe + e * q
    qe = qe / den
    out_ref[...] = -jnp.sum(jnp.abs(qe - t_ref[...]), axis=1)


def _att_call(nq, qblks, tblk, qcat, tcat, W1, b1, W2):
    def body(*refs):
        _att_body(nq, refs[0:nq], refs[nq], refs[nq + 1], refs[nq + 2], refs[nq + 3], refs[nq + 4])

    qspecs = [pl.BlockSpec((_BM, D), lambda i, j=jb: (j * _NB + i, 0)) for jb in qblks]
    return pl.pallas_call(
        body,
        grid=(_NB,),
        in_specs=qspecs + [
            pl.BlockSpec((_BM, D), lambda i, j=tblk: (j * _NB + i, 0)),
            pl.BlockSpec((D, D), lambda i: (0, 0)),
            pl.BlockSpec((1, D), lambda i: (0, 0)),
            pl.BlockSpec((D, D), lambda i: (0, 0)),
        ],
        out_specs=pl.BlockSpec((_BM,), lambda i: (i,)),
        out_shape=jax.ShapeDtypeStruct((MPAD,), jnp.float32),
    )(*([qcat] * nq), tcat, W1, b1.reshape(1, D), W2)


def _loss_body(lg_ref, out_ref):
    acc = jnp.zeros((1, D), jnp.float32)
    lane = lax.broadcasted_iota(jnp.int32, (1, D), 1)
    rows = lax.broadcasted_iota(jnp.int32, (MPAD // 64, 63), 0)
    for g in range(10):
        lg = lg_ref[g]
        z = lg[:, 0:1] - lg[:, 1:]
        ls = jnp.minimum(z, 0.0) - jnp.log1p(jnp.exp(-jnp.abs(z)))
        s = jnp.sum(jnp.where(rows < NG[g], ls, 0.0))
        acc = acc + jnp.where(lane == g, -s / (NG[g] * 63), 0.0)
    out_ref[...] = acc


def _loss_call(LG):
    out = pl.pallas_call(
        _loss_body,
        out_shape=jax.ShapeDtypeStruct((1, D), jnp.float32),
    )(LG)
    return out[0, :10]


def kernel(x, e_emb, c_emb, r_emb, W1, b1, W2, b2, Wcc1, bcc1, Wcc2, bcc2):
    A, B2h = _derive_tables(c_emb, Wcc1, bcc1)
    l8, qcat, tcat = _sc_call(e_emb, c_emb, r_emb, A, B2h, Wcc2.reshape(D),
                              x.reshape(-1))
    l8 = l8.reshape(8, MPAD)
    l5 = _att_call(2, [0, 1], 0, qcat, tcat, W1, b1, W2)
    l6 = _att_call(3, [2, 3, 4], 1, qcat, tcat, W1, b1, W2)
    LG = jnp.stack([l8[0], l8[1], l8[2], l8[3], l8[4], l5, l6,
                    l8[5], l8[6], l8[7]]).reshape(10, MPAD // 64, 64)
    return _loss_call(LG)


# speedup vs baseline: 3.9836x; 1.0174x over previous
import functools

import jax
import jax.numpy as jnp
import numpy as np
from jax import lax
from jax.experimental import pallas as pl
from jax.experimental.pallas import tpu as pltpu
from jax.experimental.pallas import tpu_sc as plsc

NV = 1000
D = 128
NW = 32
PW = 832
MPAD = NW * PW
CH = 64
NCH = PW // CH
NG = [410] * 6 + [409] * 4
DW = D // 2

_TASKS = [
    ("relu", dict(g=0, lrow=0, ops=[(3, 6), (4, 7)])),
    ("dot", dict(g=1, lrow=1, plus=[(0, 6)], c=(1, 7))),
    ("abs", dict(g=2, lrow=2, plus=[(0, 5), (2, 6)], minus=(0, 7))),
    ("abs", dict(g=3, lrow=3, plus=[(0, 4), (2, 5), (2, 6)], minus=(0, 7))),
    ("abs", dict(g=4, lrow=4, plus=[(0, 3), (2, 4), (2, 5), (2, 6)], minus=(0, 7))),
    ("q", dict(g=5, qblk=0, a=(0, 3), b=(2, 4))),
    ("q", dict(g=5, qblk=1, a=(0, 5), b=(2, 6))),
    ("t", dict(g=5, tblk=0, t=(0, 7))),
    ("q", dict(g=6, qblk=2, a=(0, 1), b=(2, 2))),
    ("q", dict(g=6, qblk=3, a=(0, 3), b=(2, 4))),
    ("q", dict(g=6, qblk=4, a=(0, 5), b=(2, 6))),
    ("t", dict(g=6, tblk=1, t=(0, 7))),
    ("dot", dict(g=7, lrow=5, plus=[(0, 5), (2, 6)], c=(1, 7))),
    ("dot", dict(g=8, lrow=6, plus=[(0, 4), (2, 5), (2, 6)], c=(1, 7))),
    ("dot", dict(g=9, lrow=7, plus=[(0, 3), (2, 4), (2, 5), (2, 6)], c=(1, 7))),
]


def _k1_body(c_ref, w_ref, b_ref, a_out, b_out):
    cc = c_ref[...]
    a_out[...] = jnp.dot(cc, w_ref[0:D, :], preferred_element_type=jnp.float32)
    b_out[...] = jnp.dot(cc, w_ref[D:2 * D, :], preferred_element_type=jnp.float32) + b_ref[...]


def _derive_tables(c_emb, Wcc1, bcc1):
    return pl.pallas_call(
        _k1_body,
        out_shape=[jax.ShapeDtypeStruct((NV, D), jnp.float32),
                   jax.ShapeDtypeStruct((NV, D), jnp.float32)],
    )(c_emb, Wcc1, bcc1.reshape(1, D))


def _sc_body(e_hbm, c_hbm, r_hbm, a_hbm, b2_hbm, w2_hbm, xf_hbm,
             l8_out, q_out, t_out,
             i0, i1, i2, i3, i4, i5, i6, b0, b1, b2, b3, b4, b5, b6, b7, b8, b9,
             xrow, lbuf, w2v, semA, semB):
    tables = [e_hbm, c_hbm, r_hbm, a_hbm, b2_hbm]
    iscr = [i0, i1, i2, i3, i4, i5, i6]
    bscr = [b0, b1, b2, b3, b4, b5, b6, b7, b8, b9]
    wid = lax.axis_index("s") * 2 + lax.axis_index("c")
    base = wid * PW
    lane = lax.iota(jnp.int32, 16)

    pltpu.sync_copy(w2_hbm, w2v)

    def stage_x(g):
        def issue(j, _):
            n_eff = jnp.minimum(wid * 13 + j, NG[g] - 1)
            pltpu.async_copy(
                xf_hbm.at[pl.ds((g + 10 * n_eff) * 512, 512)],
                xrow.at[pl.ds(j * 512, 512)], semA)
            return 0

        def drain(j, _):
            pltpu.make_async_copy(
                xf_hbm.at[pl.ds(0, 512)], xrow.at[pl.ds(0, 512)], semA).wait()
            return 0

        lax.fori_loop(0, 13, issue, 0)
        lax.fori_loop(0, 13, drain, 0)

    def extract_cols(cols):
        def ext(vb, _):
            p = vb * 16 + lane
            fb = (p >> 6) * 512 + (p & 63) * 8
            for col in cols:
                v = plsc.load_gather(xrow, [fb + col])
                iscr[col - 1][pl.ds(vb * 16, 16)] = v
            return 0

        lax.fori_loop(0, PW // 16, ext, 0, unroll=2)

    def gather(tab_slot, col, buf, c, sem):
        return pltpu.async_copy(
            tables[tab_slot].at[iscr[col - 1].at[pl.ds(c * CH, CH)]],
            buf, sem)

    def pipeline(issue, drain, compute):
        issue(jnp.int32(0), 0)

        def pair(i, _):
            cA = 2 * i
            issue(cA + 1, 1)
            drain(0)
            compute(cA, 0)
            issue(cA + 2, 0)
            drain(1)
            compute(cA + 1, 1)
            return 0

        lax.fori_loop(0, (NCH - 1) // 2, pair, 0)
        drain(0)
        compute(jnp.int32(NCH - 1), 0)

    def reduce_task(kind, spec):
        if kind == "relu":
            ops = spec["ops"]
            nplus, has_minus, has_c = len(ops), False, False
        elif kind == "abs":
            ops = spec["plus"] + [spec["minus"]]
            nplus, has_minus, has_c = len(spec["plus"]), True, False
        else:
            ops = spec["plus"] + [spec["c"]]
            nplus, has_minus, has_c = len(spec["plus"]), False, True
        w2k = [w2v[pl.ds(k * 16, 16)] for k in range(8)]
        sets = [bscr[0:len(ops)], bscr[5:5 + len(ops)]]
        sems = [semA, semB]

        def issue(c, s):
            for n, (t, col) in enumerate(ops):
                gather(t, col, sets[s][n], c, sems[s])

        def drain(s):
            for n, (t, col) in enumerate(ops):
                pltpu.make_async_copy(
                    tables[t].at[iscr[col - 1].at[pl.ds(0, CH)]],
                    sets[s][n], sems[s]).wait()

        def compute(c, s):
            bset = sets[s]

            def vblock(vb, _):
                def item(j, vec):
                    i = vb * 16 + j
                    acc = None
                    for k in range(8):
                        sl = pl.ds(k * 16, 16)
                        s_ = bset[0][i, sl]
                        for n in range(1, nplus):
                            s_ = s_ + bset[n][i, sl]
                        if has_minus:
                            v = jnp.abs(s_ - bset[nplus][i, sl])
                        elif has_c:
                            v = s_ * bset[nplus][i, sl]
                        else:
                            v = jnp.maximum(s_, 0.0) * w2k[k]
                        acc = v if acc is None else acc + v
                    tot = jnp.sum(acc)
                    return jnp.where(lane == j, tot, vec)

                vec = lax.fori_loop(0, 16, item, jnp.zeros((16,), jnp.float32),
                                    unroll=2)
                lbuf[pl.ds(c * CH + vb * 16, 16)] = -vec if kind == "abs" else vec
                return 0

            lax.fori_loop(0, CH // 16, vblock, 0)

        pipeline(issue, drain, compute)
        pltpu.sync_copy(lbuf, l8_out.at[pl.ds(spec["lrow"] * MPAD + base, PW)])

    def q_task(spec):
        off = spec["qblk"] * MPAD + base
        ta, ca = spec["a"]
        tb, cb = spec["b"]
        sets = [(b0, b1, b2), (b5, b6, b7)]
        sems = [semA, semB]

        def issue(c, s):
            gather(ta, ca, sets[s][0], c, sems[s])
            gather(tb, cb, sets[s][1], c, sems[s])

        def drain(s):
            for n, (t, col) in enumerate([spec["a"], spec["b"]]):
                pltpu.make_async_copy(
                    tables[t].at[iscr[col - 1].at[pl.ds(0, CH)]],
                    sets[s][n], sems[s]).wait()

        def compute(c, s):
            ba, bb, bq = sets[s]

            def item(i, _):
                for k in range(8):
                    sl = pl.ds(k * 16, 16)
                    bq[i, sl] = ba[i, sl] + bb[i, sl]
                return 0

            lax.fori_loop(0, CH, item, 0, unroll=2)
            pltpu.sync_copy(bq, q_out.at[pl.ds(off + c * CH, CH)])

        pipeline(issue, drain, compute)

    def t_task(spec):
        off = spec["tblk"] * MPAD + base
        tt, ct = spec["t"]
        sets = [b0, b5]
        sems = [semA, semB]

        def issue(c, s):
            gather(tt, ct, sets[s], c, sems[s])

        def drain(s):
            pltpu.make_async_copy(
                tables[tt].at[iscr[ct - 1].at[pl.ds(0, CH)]],
                sets[s], sems[s]).wait()

        def compute(c, s):
            pltpu.sync_copy(sets[s], t_out.at[pl.ds(off + c * CH, CH)])

        pipeline(issue, drain, compute)

    def task_cols(kind, spec):
        if kind == "relu":
            return [c for _, c in spec["ops"]]
        if kind == "abs":
            return [c for _, c in spec["plus"]] + [spec["minus"][1]]
        if kind == "dot":
            return [c for _, c in spec["plus"]] + [spec["c"][1]]
        if kind == "q":
            return [spec["a"][1], spec["b"][1]]
        return [spec["t"][1]]

    for g in range(10):
        gtasks = [(k, s) for k, s in _TASKS if s["g"] == g]
        stage_x(g)
        cols = sorted({c for k, s in gtasks for c in task_cols(k, s)})
        extract_cols(cols)
        for kind, spec in gtasks:
            if kind == "q":
                q_task(spec)
            elif kind == "t":
                t_task(spec)
            else:
                reduce_task(kind, spec)


def _sc_call(e_emb, c_emb, r_emb, A, B2h, w2, xf):
    mesh = plsc.VectorSubcoreMesh(core_axis_name="c", subcore_axis_name="s")
    fn = functools.partial(
        pl.kernel,
        mesh=mesh,
        compiler_params=pltpu.CompilerParams(needs_layout_passes=False),
        out_type=[
            jax.ShapeDtypeStruct((8 * MPAD,), jnp.float32),
            jax.ShapeDtypeStruct((5 * MPAD, D), jnp.float32),
            jax.ShapeDtypeStruct((2 * MPAD, D), jnp.float32),
        ],
        scratch_types=(
            [pltpu.VMEM((PW,), jnp.int32) for _ in range(7)]
            + [pltpu.VMEM((CH, D), jnp.float32) for _ in range(10)]
            + [pltpu.VMEM((13 * 512,), jnp.int32),
               pltpu.VMEM((PW,), jnp.float32),
               pltpu.VMEM((D,), jnp.float32)]
            + [pltpu.SemaphoreType.DMA,
               pltpu.SemaphoreType.DMA]
        ),
    )(_sc_body)
    return fn(e_emb, c_emb, r_emb, A, B2h, w2, xf)


_BM = 1024
_NB = MPAD // _BM


def _att_body(nq, q_refs, t_ref, w1_ref, b1_ref, w2_ref, out_ref):
    w1 = w1_ref[...].astype(jnp.bfloat16)
    w2 = w2_ref[...].astype(jnp.bfloat16)
    b1 = b1_ref[...]
    qs = [q[...] for q in q_refs]
    qb = [q.astype(jnp.bfloat16) for q in qs]
    zs = [jnp.dot(
        jnp.maximum(jnp.dot(q, w1, preferred_element_type=jnp.float32) + b1,
                    0.0).astype(jnp.bfloat16),
        w2, preferred_element_type=jnp.float32) for q in qb]
    m = zs[0]
    for z in zs[1:]:
        m = jnp.maximum(m, z)
    es = [jnp.exp(z - m) for z in zs]
    den = es[0]
    for e in es[1:]:
        den = den + e
    qe = es[0] * qs[0]
    for e, q in zip(es[1:], qs[1:]):
        qe = qe + e * q
    qe = qe / den
    out_ref[...] = -jnp.sum(jnp.abs(qe - t_ref[...].astype(jnp.float32)), axis=1)


def _att_call(nq, qblks, tblk, qcat, tcat, W1, b1, W2):
    def body(*refs):
        _att_body(nq, refs[0:nq], refs[nq], refs[nq + 1], refs[nq + 2], refs[nq + 3], refs[nq + 4])

    qspecs = [pl.BlockSpec((_BM, D), lambda i, j=jb: (j * _NB + i, 0)) for jb in qblks]
    return pl.pallas_call(
        body,
        grid=(_NB,),
        in_specs=qspecs + [
            pl.BlockSpec((_BM, D), lambda i, j=tblk: (j * _NB + i, 0)),
            pl.BlockSpec((D, D), lambda i: (0, 0)),
            pl.BlockSpec((1, D), lambda i: (0, 0)),
            pl.BlockSpec((D, D), lambda i: (0, 0)),
        ],
        out_specs=pl.BlockSpec((_BM,), lambda i: (i,)),
        out_shape=jax.ShapeDtypeStruct((MPAD,), jnp.float32),
    )(*([qcat] * nq), tcat, W1, b1.reshape(1, D), W2)


def _loss_body(lg_ref, out_ref):
    acc = jnp.zeros((1, D), jnp.float32)
    lane = lax.broadcasted_iota(jnp.int32, (1, D), 1)
    rows = lax.broadcasted_iota(jnp.int32, (MPAD // 64, 63), 0)
    for g in range(10):
        lg = lg_ref[g]
        z = lg[:, 0:1] - lg[:, 1:]
        ls = jnp.minimum(z, 0.0) - jnp.log1p(jnp.exp(-jnp.abs(z)))
        s = jnp.sum(jnp.where(rows < NG[g], ls, 0.0))
        acc = acc + jnp.where(lane == g, -s / (NG[g] * 63), 0.0)
    out_ref[...] = acc


def _loss_call(LG):
    out = pl.pallas_call(
        _loss_body,
        out_shape=jax.ShapeDtypeStruct((1, D), jnp.float32),
    )(LG)
    return out[0, :10]


def kernel(x, e_emb, c_emb, r_emb, W1, b1, W2, b2, Wcc1, bcc1, Wcc2, bcc2):
    A, B2h = _derive_tables(c_emb, Wcc1, bcc1)
    l8, qcat, tcat = _sc_call(e_emb, c_emb, r_emb, A, B2h, Wcc2.reshape(D),
                              x.reshape(-1))
    l8 = l8.reshape(8, MPAD)
    l5 = _att_call(2, [0, 1], 0, qcat, tcat, W1, b1, W2)
    l6 = _att_call(3, [2, 3, 4], 1, qcat, tcat, W1, b1, W2)
    LG = jnp.stack([l8[0], l8[1], l8[2], l8[3], l8[4], l5, l6,
                    l8[5], l8[6], l8[7]]).reshape(10, MPAD // 64, 64)
    return _loss_call(LG)


# speedup vs baseline: 4.3051x; 1.0807x over previous
import functools

import jax
import jax.numpy as jnp
import numpy as np
from jax import lax
from jax.experimental import pallas as pl
from jax.experimental.pallas import tpu as pltpu
from jax.experimental.pallas import tpu_sc as plsc

NV = 1000
D = 128
NW = 32
PW = 832
MPAD = NW * PW
CH = 64
NCH = PW // CH
NG = [410] * 6 + [409] * 4
DW = D // 2

_TASKS = [
    ("relu", dict(g=0, lrow=0, ops=[(3, 6), (4, 7)])),
    ("dot", dict(g=1, lrow=1, plus=[(0, 6)], c=(1, 7))),
    ("abs", dict(g=2, lrow=2, plus=[(0, 5), (2, 6)], minus=(0, 7))),
    ("abs", dict(g=3, lrow=3, plus=[(0, 4), (2, 5), (2, 6)], minus=(0, 7))),
    ("abs", dict(g=4, lrow=4, plus=[(0, 3), (2, 4), (2, 5), (2, 6)], minus=(0, 7))),
    ("q", dict(g=5, qblk=0, a=(0, 3), b=(2, 4))),
    ("q", dict(g=5, qblk=1, a=(0, 5), b=(2, 6))),
    ("t", dict(g=5, tblk=0, t=(0, 7))),
    ("q", dict(g=6, qblk=2, a=(0, 1), b=(2, 2))),
    ("q", dict(g=6, qblk=3, a=(0, 3), b=(2, 4))),
    ("q", dict(g=6, qblk=4, a=(0, 5), b=(2, 6))),
    ("t", dict(g=6, tblk=1, t=(0, 7))),
    ("dot", dict(g=7, lrow=5, plus=[(0, 5), (2, 6)], c=(1, 7))),
    ("dot", dict(g=8, lrow=6, plus=[(0, 4), (2, 5), (2, 6)], c=(1, 7))),
    ("dot", dict(g=9, lrow=7, plus=[(0, 3), (2, 4), (2, 5), (2, 6)], c=(1, 7))),
]


def _k1_body(c_ref, w_ref, b_ref, a_out, b_out):
    cc = c_ref[...]
    a_out[...] = jnp.dot(cc, w_ref[0:D, :], preferred_element_type=jnp.float32)
    b_out[...] = jnp.dot(cc, w_ref[D:2 * D, :], preferred_element_type=jnp.float32) + b_ref[...]


def _derive_tables(c_emb, Wcc1, bcc1):
    return pl.pallas_call(
        _k1_body,
        out_shape=[jax.ShapeDtypeStruct((NV, D), jnp.float32),
                   jax.ShapeDtypeStruct((NV, D), jnp.float32)],
    )(c_emb, Wcc1, bcc1.reshape(1, D))


def _sc_engine(groups, tables, w2_hbm, xf_hbm, l8_out, q_out, t_out,
               iscr, bscr, xrow, lbuf, w2v, semA, semB):
    wid = lax.axis_index("s") * 2 + lax.axis_index("c")
    base = wid * PW
    lane = lax.iota(jnp.int32, 16)

    if w2_hbm is not None:
        pltpu.sync_copy(w2_hbm, w2v)

    def stage_x(g):
        def issue(j, _):
            n_eff = jnp.minimum(wid * 13 + j, NG[g] - 1)
            pltpu.async_copy(
                xf_hbm.at[pl.ds((g + 10 * n_eff) * 512, 512)],
                xrow.at[pl.ds(j * 512, 512)], semA)
            return 0

        def drain(j, _):
            pltpu.make_async_copy(
                xf_hbm.at[pl.ds(0, 512)], xrow.at[pl.ds(0, 512)], semA).wait()
            return 0

        lax.fori_loop(0, 13, issue, 0)
        lax.fori_loop(0, 13, drain, 0)

    def extract_cols(cols):
        def ext(vb, _):
            p = vb * 16 + lane
            fb = (p >> 6) * 512 + (p & 63) * 8
            for col in cols:
                v = plsc.load_gather(xrow, [fb + col])
                iscr[col - 1][pl.ds(vb * 16, 16)] = v
            return 0

        lax.fori_loop(0, PW // 16, ext, 0, unroll=2)

    def gather(tab_slot, col, buf, c, sem):
        return pltpu.async_copy(
            tables[tab_slot].at[iscr[col - 1].at[pl.ds(c * CH, CH)]],
            buf, sem)

    def pipeline(issue, drain, compute):
        issue(jnp.int32(0), 0)

        def pair(i, _):
            cA = 2 * i
            issue(cA + 1, 1)
            drain(0)
            compute(cA, 0)
            issue(cA + 2, 0)
            drain(1)
            compute(cA + 1, 1)
            return 0

        lax.fori_loop(0, (NCH - 1) // 2, pair, 0)
        drain(0)
        compute(jnp.int32(NCH - 1), 0)

    def reduce_task(kind, spec):
        if kind == "relu":
            ops = spec["ops"]
            nplus, has_minus, has_c = len(ops), False, False
        elif kind == "abs":
            ops = spec["plus"] + [spec["minus"]]
            nplus, has_minus, has_c = len(spec["plus"]), True, False
        else:
            ops = spec["plus"] + [spec["c"]]
            nplus, has_minus, has_c = len(spec["plus"]), False, True
        w2k = [w2v[pl.ds(k * 16, 16)] for k in range(8)]
        sets = [bscr[0:len(ops)], bscr[5:5 + len(ops)]]
        sems = [semA, semB]

        def issue(c, s):
            for n, (t, col) in enumerate(ops):
                gather(t, col, sets[s][n], c, sems[s])

        def drain(s):
            for n, (t, col) in enumerate(ops):
                pltpu.make_async_copy(
                    tables[t].at[iscr[col - 1].at[pl.ds(0, CH)]],
                    sets[s][n], sems[s]).wait()

        def compute(c, s):
            bset = sets[s]

            def vblock(vb, _):
                def item(j, vec):
                    i = vb * 16 + j
                    acc = None
                    for k in range(8):
                        sl = pl.ds(k * 16, 16)
                        s_ = bset[0][i, sl]
                        for n in range(1, nplus):
                            s_ = s_ + bset[n][i, sl]
                        if has_minus:
                            v = jnp.abs(s_ - bset[nplus][i, sl])
                        elif has_c:
                            v = s_ * bset[nplus][i, sl]
                        else:
                            v = jnp.maximum(s_, 0.0) * w2k[k]
                        acc = v if acc is None else acc + v
                    tot = jnp.sum(acc)
                    return jnp.where(lane == j, tot, vec)

                vec = lax.fori_loop(0, 16, item, jnp.zeros((16,), jnp.float32),
                                    unroll=2)
                lbuf[pl.ds(c * CH + vb * 16, 16)] = -vec if kind == "abs" else vec
                return 0

            lax.fori_loop(0, CH // 16, vblock, 0)

        pipeline(issue, drain, compute)
        pltpu.sync_copy(lbuf, l8_out.at[pl.ds(spec["lrow"] * MPAD + base, PW)])

    def q_task(spec):
        off = spec["qblk"] * MPAD + base
        ta, ca = spec["a"]
        tb, cb = spec["b"]
        sets = [(bscr[0], bscr[1], bscr[2]), (bscr[5], bscr[6], bscr[7])]
        sems = [semA, semB]

        def issue(c, s):
            gather(ta, ca, sets[s][0], c, sems[s])
            gather(tb, cb, sets[s][1], c, sems[s])

        def drain(s):
            for n, (t, col) in enumerate([spec["a"], spec["b"]]):
                pltpu.make_async_copy(
                    tables[t].at[iscr[col - 1].at[pl.ds(0, CH)]],
                    sets[s][n], sems[s]).wait()

        def compute(c, s):
            ba, bb, bq = sets[s]

            def item(i, _):
                for k in range(8):
                    sl = pl.ds(k * 16, 16)
                    bq[i, sl] = ba[i, sl] + bb[i, sl]
                return 0

            lax.fori_loop(0, CH, item, 0, unroll=2)
            pltpu.sync_copy(bq, q_out.at[pl.ds(off + c * CH, CH)])

        pipeline(issue, drain, compute)

    def t_task(spec):
        off = spec["tblk"] * MPAD + base
        tt, ct = spec["t"]
        sets = [bscr[0], bscr[5]]
        sems = [semA, semB]

        def issue(c, s):
            gather(tt, ct, sets[s], c, sems[s])

        def drain(s):
            pltpu.make_async_copy(
                tables[tt].at[iscr[ct - 1].at[pl.ds(0, CH)]],
                sets[s], sems[s]).wait()

        def compute(c, s):
            pltpu.sync_copy(sets[s], t_out.at[pl.ds(off + c * CH, CH)])

        pipeline(issue, drain, compute)

    def task_cols(kind, spec):
        if kind == "relu":
            return [c for _, c in spec["ops"]]
        if kind == "abs":
            return [c for _, c in spec["plus"]] + [spec["minus"][1]]
        if kind == "dot":
            return [c for _, c in spec["plus"]] + [spec["c"][1]]
        if kind == "q":
            return [spec["a"][1], spec["b"][1]]
        return [spec["t"][1]]

    for g in groups:
        gtasks = [(k, s) for k, s in _TASKS if s["g"] == g]
        stage_x(g)
        cols = sorted({c for k, s in gtasks for c in task_cols(k, s)})
        extract_cols(cols)
        for kind, spec in gtasks:
            if kind == "q":
                q_task(spec)
            elif kind == "t":
                t_task(spec)
            else:
                reduce_task(kind, spec)


def _sc_body_qt(e_hbm, r_hbm, xf_hbm, q_out, t_out, *scr):
    _sc_engine([5, 6], [e_hbm, None, r_hbm, None, None], None, xf_hbm,
               None, q_out, t_out, list(scr[0:7]), list(scr[7:17]),
               scr[17], scr[18], scr[19], scr[20], scr[21])


def _sc_body_red(e_hbm, c_hbm, r_hbm, a_hbm, b2_hbm, w2_hbm, xf_hbm,
                 l8_out, *scr):
    _sc_engine([0, 1, 2, 3, 4, 7, 8, 9], [e_hbm, c_hbm, r_hbm, a_hbm, b2_hbm],
               w2_hbm, xf_hbm, l8_out, None, None, list(scr[0:7]),
               list(scr[7:17]), scr[17], scr[18], scr[19], scr[20], scr[21])


_SC_SCRATCH = (
    [pltpu.VMEM((PW,), jnp.int32) for _ in range(7)]
    + [pltpu.VMEM((CH, D), jnp.float32) for _ in range(10)]
    + [pltpu.VMEM((13 * 512,), jnp.int32),
       pltpu.VMEM((PW,), jnp.float32),
       pltpu.VMEM((D,), jnp.float32)]
    + [pltpu.SemaphoreType.DMA,
       pltpu.SemaphoreType.DMA]
)


def _sc_call_qt(e_emb, r_emb, xf):
    mesh = plsc.VectorSubcoreMesh(core_axis_name="c", subcore_axis_name="s")
    fn = functools.partial(
        pl.kernel,
        mesh=mesh,
        compiler_params=pltpu.CompilerParams(needs_layout_passes=False),
        out_type=[
            jax.ShapeDtypeStruct((5 * MPAD, D), jnp.float32),
            jax.ShapeDtypeStruct((2 * MPAD, D), jnp.float32),
        ],
        scratch_types=_SC_SCRATCH,
    )(_sc_body_qt)
    return fn(e_emb, r_emb, xf)


def _sc_call_red(e_emb, c_emb, r_emb, A, B2h, w2, xf):
    mesh = plsc.VectorSubcoreMesh(core_axis_name="c", subcore_axis_name="s")
    fn = functools.partial(
        pl.kernel,
        mesh=mesh,
        compiler_params=pltpu.CompilerParams(needs_layout_passes=False),
        out_type=[
            jax.ShapeDtypeStruct((8 * MPAD,), jnp.float32),
        ],
        scratch_types=_SC_SCRATCH,
    )(_sc_body_red)
    return fn(e_emb, c_emb, r_emb, A, B2h, w2, xf)


_BM = 1024
_NB = MPAD // _BM


def _att_body(nq, q_refs, t_ref, w1_ref, b1_ref, w2_ref, out_ref):
    w1 = w1_ref[...].astype(jnp.bfloat16)
    w2 = w2_ref[...].astype(jnp.bfloat16)
    b1 = b1_ref[...]
    qs = [q[...] for q in q_refs]
    qb = [q.astype(jnp.bfloat16) for q in qs]
    zs = [jnp.dot(
        jnp.maximum(jnp.dot(q, w1, preferred_element_type=jnp.float32) + b1,
                    0.0).astype(jnp.bfloat16),
        w2, preferred_element_type=jnp.float32) for q in qb]
    m = zs[0]
    for z in zs[1:]:
        m = jnp.maximum(m, z)
    es = [jnp.exp(z - m) for z in zs]
    den = es[0]
    for e in es[1:]:
        den = den + e
    qe = es[0] * qs[0]
    for e, q in zip(es[1:], qs[1:]):
        qe = qe + e * q
    qe = qe / den
    out_ref[...] = -jnp.sum(jnp.abs(qe - t_ref[...].astype(jnp.float32)), axis=1)


def _att_call(nq, qblks, tblk, qcat, tcat, W1, b1, W2):
    def body(*refs):
        _att_body(nq, refs[0:nq], refs[nq], refs[nq + 1], refs[nq + 2], refs[nq + 3], refs[nq + 4])

    qspecs = [pl.BlockSpec((_BM, D), lambda i, j=jb: (j * _NB + i, 0)) for jb in qblks]
    return pl.pallas_call(
        body,
        grid=(_NB,),
        in_specs=qspecs + [
            pl.BlockSpec((_BM, D), lambda i, j=tblk: (j * _NB + i, 0)),
            pl.BlockSpec((D, D), lambda i: (0, 0)),
            pl.BlockSpec((1, D), lambda i: (0, 0)),
            pl.BlockSpec((D, D), lambda i: (0, 0)),
        ],
        out_specs=pl.BlockSpec((_BM,), lambda i: (i,)),
        out_shape=jax.ShapeDtypeStruct((MPAD,), jnp.float32),
    )(*([qcat] * nq), tcat, W1, b1.reshape(1, D), W2)


def _loss_body(lg_ref, out_ref):
    acc = jnp.zeros((1, D), jnp.float32)
    lane = lax.broadcasted_iota(jnp.int32, (1, D), 1)
    rows = lax.broadcasted_iota(jnp.int32, (MPAD // 64, 63), 0)
    for g in range(10):
        lg = lg_ref[g]
        z = lg[:, 0:1] - lg[:, 1:]
        ls = jnp.minimum(z, 0.0) - jnp.log1p(jnp.exp(-jnp.abs(z)))
        s = jnp.sum(jnp.where(rows < NG[g], ls, 0.0))
        acc = acc + jnp.where(lane == g, -s / (NG[g] * 63), 0.0)
    out_ref[...] = acc


def _loss_call(LG):
    out = pl.pallas_call(
        _loss_body,
        out_shape=jax.ShapeDtypeStruct((1, D), jnp.float32),
    )(LG)
    return out[0, :10]


def kernel(x, e_emb, c_emb, r_emb, W1, b1, W2, b2, Wcc1, bcc1, Wcc2, bcc2):
    A, B2h = _derive_tables(c_emb, Wcc1, bcc1)
    xf = x.reshape(-1)
    qcat, tcat = _sc_call_qt(e_emb, r_emb, xf)
    (l8,) = _sc_call_red(e_emb, c_emb, r_emb, A, B2h, Wcc2.reshape(D), xf)
    l8 = l8.reshape(8, MPAD)
    l5 = _att_call(2, [0, 1], 0, qcat, tcat, W1, b1, W2)
    l6 = _att_call(3, [2, 3, 4], 1, qcat, tcat, W1, b1, W2)
    LG = jnp.stack([l8[0], l8[1], l8[2], l8[3], l8[4], l5, l6,
                    l8[5], l8[6], l8[7]]).reshape(10, MPAD // 64, 64)
    return _loss_call(LG)


# speedup vs baseline: 4.3121x; 1.0016x over previous
import functools

import jax
import jax.numpy as jnp
import numpy as np
from jax import lax
from jax.experimental import pallas as pl
from jax.experimental.pallas import tpu as pltpu
from jax.experimental.pallas import tpu_sc as plsc

NV = 1000
D = 128
NW = 32
PW = 832
MPAD = NW * PW
CH = 64
NCH = PW // CH
NG = [410] * 6 + [409] * 4
DW = D // 2

_TASKS = [
    ("relu", dict(g=0, lrow=0, ops=[(3, 6), (4, 7)])),
    ("dot", dict(g=1, lrow=1, plus=[(0, 6)], c=(1, 7))),
    ("abs", dict(g=2, lrow=2, plus=[(0, 5), (2, 6)], minus=(0, 7))),
    ("abs", dict(g=3, lrow=3, plus=[(0, 4), (2, 5), (2, 6)], minus=(0, 7))),
    ("abs", dict(g=4, lrow=4, plus=[(0, 3), (2, 4), (2, 5), (2, 6)], minus=(0, 7))),
    ("q", dict(g=5, qblk=0, a=(0, 3), b=(2, 4))),
    ("q", dict(g=5, qblk=1, a=(0, 5), b=(2, 6))),
    ("t", dict(g=5, tblk=0, t=(0, 7))),
    ("q", dict(g=6, qblk=2, a=(0, 1), b=(2, 2))),
    ("q", dict(g=6, qblk=3, a=(0, 3), b=(2, 4))),
    ("q", dict(g=6, qblk=4, a=(0, 5), b=(2, 6))),
    ("t", dict(g=6, tblk=1, t=(0, 7))),
    ("dot", dict(g=7, lrow=5, plus=[(0, 5), (2, 6)], c=(1, 7))),
    ("dot", dict(g=8, lrow=6, plus=[(0, 4), (2, 5), (2, 6)], c=(1, 7))),
    ("dot", dict(g=9, lrow=7, plus=[(0, 3), (2, 4), (2, 5), (2, 6)], c=(1, 7))),
]


def _k1_body(c_ref, w_ref, b_ref, a_out, b_out):
    cc = c_ref[...]
    a_out[...] = jnp.dot(cc, w_ref[0:D, :], preferred_element_type=jnp.float32)
    b_out[...] = jnp.dot(cc, w_ref[D:2 * D, :], preferred_element_type=jnp.float32) + b_ref[...]


def _derive_tables(c_emb, Wcc1, bcc1):
    return pl.pallas_call(
        _k1_body,
        out_shape=[jax.ShapeDtypeStruct((NV, D), jnp.float32),
                   jax.ShapeDtypeStruct((NV, D), jnp.float32)],
    )(c_emb, Wcc1, bcc1.reshape(1, D))


def _sc_engine(groups, tables, w2_hbm, xf_hbm, l8_out, q_out, t_out,
               iscr, bscr, xrow, lbuf, w2v, semA, semB,
               row_of=lambda g, n: g + 10 * n):
    wid = lax.axis_index("s") * 2 + lax.axis_index("c")
    base = wid * PW
    lane = lax.iota(jnp.int32, 16)

    if w2_hbm is not None:
        pltpu.sync_copy(w2_hbm, w2v)

    def stage_x(g):
        def issue(j, _):
            n_eff = jnp.minimum(wid * 13 + j, NG[g] - 1)
            pltpu.async_copy(
                xf_hbm.at[pl.ds(row_of(g, n_eff) * 512, 512)],
                xrow.at[pl.ds(j * 512, 512)], semA)
            return 0

        def drain(j, _):
            pltpu.make_async_copy(
                xf_hbm.at[pl.ds(0, 512)], xrow.at[pl.ds(0, 512)], semA).wait()
            return 0

        lax.fori_loop(0, 13, issue, 0)
        lax.fori_loop(0, 13, drain, 0)

    def extract_cols(cols):
        def ext(vb, _):
            p = vb * 16 + lane
            fb = (p >> 6) * 512 + (p & 63) * 8
            for col in cols:
                v = plsc.load_gather(xrow, [fb + col])
                iscr[col - 1][pl.ds(vb * 16, 16)] = v
            return 0

        lax.fori_loop(0, PW // 16, ext, 0, unroll=2)

    def gather(tab_slot, col, buf, c, sem):
        return pltpu.async_copy(
            tables[tab_slot].at[iscr[col - 1].at[pl.ds(c * CH, CH)]],
            buf, sem)

    def pipeline(issue, drain, compute):
        issue(jnp.int32(0), 0)

        def pair(i, _):
            cA = 2 * i
            issue(cA + 1, 1)
            drain(0)
            compute(cA, 0)
            issue(cA + 2, 0)
            drain(1)
            compute(cA + 1, 1)
            return 0

        lax.fori_loop(0, (NCH - 1) // 2, pair, 0)
        drain(0)
        compute(jnp.int32(NCH - 1), 0)

    def reduce_task(kind, spec):
        if kind == "relu":
            ops = spec["ops"]
            nplus, has_minus, has_c = len(ops), False, False
        elif kind == "abs":
            ops = spec["plus"] + [spec["minus"]]
            nplus, has_minus, has_c = len(spec["plus"]), True, False
        else:
            ops = spec["plus"] + [spec["c"]]
            nplus, has_minus, has_c = len(spec["plus"]), False, True
        w2k = [w2v[pl.ds(k * 16, 16)] for k in range(8)]
        sets = [bscr[0:len(ops)], bscr[5:5 + len(ops)]]
        sems = [semA, semB]

        def issue(c, s):
            for n, (t, col) in enumerate(ops):
                gather(t, col, sets[s][n], c, sems[s])

        def drain(s):
            for n, (t, col) in enumerate(ops):
                pltpu.make_async_copy(
                    tables[t].at[iscr[col - 1].at[pl.ds(0, CH)]],
                    sets[s][n], sems[s]).wait()

        def compute(c, s):
            bset = sets[s]

            def vblock(vb, _):
                def item(j, vec):
                    i = vb * 16 + j
                    acc = None
                    for k in range(8):
                        sl = pl.ds(k * 16, 16)
                        s_ = bset[0][i, sl]
                        for n in range(1, nplus):
                            s_ = s_ + bset[n][i, sl]
                        if has_minus:
                            v = jnp.abs(s_ - bset[nplus][i, sl])
                        elif has_c:
                            v = s_ * bset[nplus][i, sl]
                        else:
                            v = jnp.maximum(s_, 0.0) * w2k[k]
                        acc = v if acc is None else acc + v
                    tot = jnp.sum(acc)
                    return jnp.where(lane == j, tot, vec)

                vec = lax.fori_loop(0, 16, item, jnp.zeros((16,), jnp.float32),
                                    unroll=2)
                lbuf[pl.ds(c * CH + vb * 16, 16)] = -vec if kind == "abs" else vec
                return 0

            lax.fori_loop(0, CH // 16, vblock, 0)

        pipeline(issue, drain, compute)
        pltpu.sync_copy(lbuf, l8_out.at[pl.ds(spec["lrow"] * MPAD + base, PW)])

    def q_task(spec):
        off = spec["qblk"] * MPAD + base
        ta, ca = spec["a"]
        tb, cb = spec["b"]
        sets = [(bscr[0], bscr[1], bscr[2]), (bscr[5], bscr[6], bscr[7])]
        sems = [semA, semB]

        def issue(c, s):
            gather(ta, ca, sets[s][0], c, sems[s])
            gather(tb, cb, sets[s][1], c, sems[s])

        def drain(s):
            for n, (t, col) in enumerate([spec["a"], spec["b"]]):
                pltpu.make_async_copy(
                    tables[t].at[iscr[col - 1].at[pl.ds(0, CH)]],
                    sets[s][n], sems[s]).wait()

        def compute(c, s):
            ba, bb, bq = sets[s]

            def item(i, _):
                for k in range(8):
                    sl = pl.ds(k * 16, 16)
                    bq[i, sl] = ba[i, sl] + bb[i, sl]
                return 0

            lax.fori_loop(0, CH, item, 0, unroll=2)
            pltpu.sync_copy(bq, q_out.at[pl.ds(off + c * CH, CH)])

        pipeline(issue, drain, compute)

    def t_task(spec):
        off = spec["tblk"] * MPAD + base
        tt, ct = spec["t"]
        sets = [bscr[0], bscr[5]]
        sems = [semA, semB]

        def issue(c, s):
            gather(tt, ct, sets[s], c, sems[s])

        def drain(s):
            pltpu.make_async_copy(
                tables[tt].at[iscr[ct - 1].at[pl.ds(0, CH)]],
                sets[s], sems[s]).wait()

        def compute(c, s):
            pltpu.sync_copy(sets[s], t_out.at[pl.ds(off + c * CH, CH)])

        pipeline(issue, drain, compute)

    def task_cols(kind, spec):
        if kind == "relu":
            return [c for _, c in spec["ops"]]
        if kind == "abs":
            return [c for _, c in spec["plus"]] + [spec["minus"][1]]
        if kind == "dot":
            return [c for _, c in spec["plus"]] + [spec["c"][1]]
        if kind == "q":
            return [spec["a"][1], spec["b"][1]]
        return [spec["t"][1]]

    for g in groups:
        gtasks = [(k, s) for k, s in _TASKS if s["g"] == g]
        stage_x(g)
        cols = sorted({c for k, s in gtasks for c in task_cols(k, s)})
        extract_cols(cols)
        for kind, spec in gtasks:
            if kind == "q":
                q_task(spec)
            elif kind == "t":
                t_task(spec)
            else:
                reduce_task(kind, spec)


def _sc_body_qt(e_hbm, r_hbm, xf_hbm, q_out, t_out, *scr):
    _sc_engine([5, 6], [e_hbm, None, r_hbm, None, None], None, xf_hbm,
               None, q_out, t_out, list(scr[0:7]), list(scr[7:17]),
               scr[17], scr[18], scr[19], scr[20], scr[21],
               row_of=lambda g, n: (0 if g == 5 else 410) + n)


def _sc_body_red(e_hbm, c_hbm, r_hbm, a_hbm, b2_hbm, w2_hbm, xf_hbm,
                 l8_out, *scr):
    _sc_engine([0, 1, 2, 3, 4, 7, 8, 9], [e_hbm, c_hbm, r_hbm, a_hbm, b2_hbm],
               w2_hbm, xf_hbm, l8_out, None, None, list(scr[0:7]),
               list(scr[7:17]), scr[17], scr[18], scr[19], scr[20], scr[21])


_SC_SCRATCH = (
    [pltpu.VMEM((PW,), jnp.int32) for _ in range(7)]
    + [pltpu.VMEM((CH, D), jnp.float32) for _ in range(10)]
    + [pltpu.VMEM((13 * 512,), jnp.int32),
       pltpu.VMEM((PW,), jnp.float32),
       pltpu.VMEM((D,), jnp.float32)]
    + [pltpu.SemaphoreType.DMA,
       pltpu.SemaphoreType.DMA]
)


def _sc_call_qt(e_emb, r_emb, xf):
    mesh = plsc.VectorSubcoreMesh(core_axis_name="c", subcore_axis_name="s")
    fn = functools.partial(
        pl.kernel,
        mesh=mesh,
        compiler_params=pltpu.CompilerParams(needs_layout_passes=False),
        out_type=[
            jax.ShapeDtypeStruct((5 * MPAD, D), jnp.float32),
            jax.ShapeDtypeStruct((2 * MPAD, D), jnp.float32),
        ],
        scratch_types=_SC_SCRATCH,
    )(_sc_body_qt)
    return fn(e_emb, r_emb, xf)


def _sc_call_red(e_emb, c_emb, r_emb, A, B2h, w2, xf):
    mesh = plsc.VectorSubcoreMesh(core_axis_name="c", subcore_axis_name="s")
    fn = functools.partial(
        pl.kernel,
        mesh=mesh,
        compiler_params=pltpu.CompilerParams(needs_layout_passes=False),
        out_type=[
            jax.ShapeDtypeStruct((8 * MPAD,), jnp.float32),
        ],
        scratch_types=_SC_SCRATCH,
    )(_sc_body_red)
    return fn(e_emb, c_emb, r_emb, A, B2h, w2, xf)


_BM = 1024
_NB = MPAD // _BM


def _att_body(nq, q_refs, t_ref, w1_ref, b1_ref, w2_ref, out_ref):
    w1 = w1_ref[...].astype(jnp.bfloat16)
    w2 = w2_ref[...].astype(jnp.bfloat16)
    b1 = b1_ref[...]
    qs = [q[...] for q in q_refs]
    qb = [q.astype(jnp.bfloat16) for q in qs]
    zs = [jnp.dot(
        jnp.maximum(jnp.dot(q, w1, preferred_element_type=jnp.float32) + b1,
                    0.0).astype(jnp.bfloat16),
        w2, preferred_element_type=jnp.float32) for q in qb]
    m = zs[0]
    for z in zs[1:]:
        m = jnp.maximum(m, z)
    es = [jnp.exp(z - m) for z in zs]
    den = es[0]
    for e in es[1:]:
        den = den + e
    qe = es[0] * qs[0]
    for e, q in zip(es[1:], qs[1:]):
        qe = qe + e * q
    qe = qe / den
    out_ref[...] = -jnp.sum(jnp.abs(qe - t_ref[...].astype(jnp.float32)), axis=1)


def _att_call(nq, qblks, tblk, qcat, tcat, W1, b1, W2):
    def body(*refs):
        _att_body(nq, refs[0:nq], refs[nq], refs[nq + 1], refs[nq + 2], refs[nq + 3], refs[nq + 4])

    qspecs = [pl.BlockSpec((_BM, D), lambda i, j=jb: (j * _NB + i, 0)) for jb in qblks]
    return pl.pallas_call(
        body,
        grid=(_NB,),
        in_specs=qspecs + [
            pl.BlockSpec((_BM, D), lambda i, j=tblk: (j * _NB + i, 0)),
            pl.BlockSpec((D, D), lambda i: (0, 0)),
            pl.BlockSpec((1, D), lambda i: (0, 0)),
            pl.BlockSpec((D, D), lambda i: (0, 0)),
        ],
        out_specs=pl.BlockSpec((_BM,), lambda i: (i,)),
        out_shape=jax.ShapeDtypeStruct((MPAD,), jnp.float32),
    )(*([qcat] * nq), tcat, W1, b1.reshape(1, D), W2)


def _loss_body(lg_ref, out_ref):
    acc = jnp.zeros((1, D), jnp.float32)
    lane = lax.broadcasted_iota(jnp.int32, (1, D), 1)
    rows = lax.broadcasted_iota(jnp.int32, (MPAD // 64, 63), 0)
    for g in range(10):
        lg = lg_ref[g]
        z = lg[:, 0:1] - lg[:, 1:]
        ls = jnp.minimum(z, 0.0) - jnp.log1p(jnp.exp(-jnp.abs(z)))
        s = jnp.sum(jnp.where(rows < NG[g], ls, 0.0))
        acc = acc + jnp.where(lane == g, -s / (NG[g] * 63), 0.0)
    out_ref[...] = acc


def _loss_call(LG):
    out = pl.pallas_call(
        _loss_body,
        out_shape=jax.ShapeDtypeStruct((1, D), jnp.float32),
    )(LG)
    return out[0, :10]


def kernel(x, e_emb, c_emb, r_emb, W1, b1, W2, b2, Wcc1, bcc1, Wcc2, bcc2):
    A, B2h = _derive_tables(c_emb, Wcc1, bcc1)
    xf = x.reshape(-1)
    xqt = jnp.concatenate([x[5::10].reshape(-1), x[6::10].reshape(-1)])
    qcat, tcat = _sc_call_qt(e_emb, r_emb, xqt)
    (l8,) = _sc_call_red(e_emb, c_emb, r_emb, A, B2h, Wcc2.reshape(D), xf)
    l8 = l8.reshape(8, MPAD)
    l5 = _att_call(2, [0, 1], 0, qcat, tcat, W1, b1, W2)
    l6 = _att_call(3, [2, 3, 4], 1, qcat, tcat, W1, b1, W2)
    LG = jnp.stack([l8[0], l8[1], l8[2], l8[3], l8[4], l5, l6,
                    l8[5], l8[6], l8[7]]).reshape(10, MPAD // 64, 64)
    return _loss_call(LG)


# speedup vs baseline: 4.6488x; 1.0781x over previous
import functools

import jax
import jax.numpy as jnp
import numpy as np
from jax import lax
from jax.experimental import pallas as pl
from jax.experimental.pallas import tpu as pltpu
from jax.experimental.pallas import tpu_sc as plsc

NV = 1000
D = 128
NW = 32
PW = 832
MPAD = NW * PW
CH = 64
NCH = PW // CH
NG = [410] * 6 + [409] * 4
DW = D // 2

_TASKS = [
    ("relu", dict(g=0, lrow=0, ops=[(3, 6), (4, 7)])),
    ("dot", dict(g=1, lrow=1, plus=[(0, 6)], c=(1, 7))),
    ("abs", dict(g=2, lrow=2, plus=[(0, 5), (2, 6)], minus=(0, 7))),
    ("abs", dict(g=3, lrow=3, plus=[(0, 4), (2, 5), (2, 6)], minus=(0, 7))),
    ("abs", dict(g=4, lrow=4, plus=[(0, 3), (2, 4), (2, 5), (2, 6)], minus=(0, 7))),
    ("q", dict(g=5, qblk=0, a=(0, 3), b=(2, 4))),
    ("q", dict(g=5, qblk=1, a=(0, 5), b=(2, 6))),
    ("t", dict(g=5, tblk=0, t=(0, 7))),
    ("q", dict(g=6, qblk=2, a=(0, 1), b=(2, 2))),
    ("q", dict(g=6, qblk=3, a=(0, 3), b=(2, 4))),
    ("q", dict(g=6, qblk=4, a=(0, 5), b=(2, 6))),
    ("t", dict(g=6, tblk=1, t=(0, 7))),
    ("dot", dict(g=7, lrow=5, plus=[(0, 5), (2, 6)], c=(1, 7))),
    ("dot", dict(g=8, lrow=6, plus=[(0, 4), (2, 5), (2, 6)], c=(1, 7))),
    ("dot", dict(g=9, lrow=7, plus=[(0, 3), (2, 4), (2, 5), (2, 6)], c=(1, 7))),
]


def _k1_body(c_ref, w_ref, b_ref, a_out, b_out):
    cc = c_ref[...]
    a_out[...] = jnp.dot(cc, w_ref[0:D, :], preferred_element_type=jnp.float32)
    b_out[...] = jnp.dot(cc, w_ref[D:2 * D, :], preferred_element_type=jnp.float32) + b_ref[...]


def _derive_tables(c_emb, Wcc1, bcc1):
    return pl.pallas_call(
        _k1_body,
        out_shape=[jax.ShapeDtypeStruct((NV, D), jnp.float32),
                   jax.ShapeDtypeStruct((NV, D), jnp.float32)],
    )(c_emb, Wcc1, bcc1.reshape(1, D))


def _sc_engine(groups, tables, w2_hbm, xf_hbm, l8_out, q_out, t_out,
               iscr, bscr, xrow, lbuf, w2v, semA, semB,
               row_of=lambda g, n: g + 10 * n):
    wid = lax.axis_index("s") * 2 + lax.axis_index("c")
    base = wid * PW
    lane = lax.iota(jnp.int32, 16)

    if w2_hbm is not None:
        pltpu.sync_copy(w2_hbm, w2v)

    def stage_x(g):
        def issue(j, _):
            n_eff = jnp.minimum(wid * 13 + j, NG[g] - 1)
            pltpu.async_copy(
                xf_hbm.at[pl.ds(row_of(g, n_eff) * 512, 512)],
                xrow.at[pl.ds(j * 512, 512)], semA)
            return 0

        def drain(j, _):
            pltpu.make_async_copy(
                xf_hbm.at[pl.ds(0, 512)], xrow.at[pl.ds(0, 512)], semA).wait()
            return 0

        lax.fori_loop(0, 13, issue, 0)
        lax.fori_loop(0, 13, drain, 0)

    def extract_cols(cols):
        def ext(vb, _):
            p = vb * 16 + lane
            fb = (p >> 6) * 512 + (p & 63) * 8
            for col in cols:
                v = plsc.load_gather(xrow, [fb + col])
                iscr[col - 1][pl.ds(vb * 16, 16)] = v
            return 0

        lax.fori_loop(0, PW // 16, ext, 0, unroll=2)

    def gather(tab_slot, col, buf, c, sem):
        return pltpu.async_copy(
            tables[tab_slot].at[iscr[col - 1].at[pl.ds(c * CH, CH)]],
            buf, sem)

    def pipeline(issue, drain, compute):
        issue(jnp.int32(0), 0)

        def pair(i, _):
            cA = 2 * i
            issue(cA + 1, 1)
            drain(0)
            compute(cA, 0)
            issue(cA + 2, 0)
            drain(1)
            compute(cA + 1, 1)
            return 0

        lax.fori_loop(0, (NCH - 1) // 2, pair, 0)
        drain(0)
        compute(jnp.int32(NCH - 1), 0)

    def reduce_task(kind, spec):
        if kind == "relu":
            ops = spec["ops"]
            nplus, has_minus, has_c = len(ops), False, False
        elif kind == "abs":
            ops = spec["plus"] + [spec["minus"]]
            nplus, has_minus, has_c = len(spec["plus"]), True, False
        else:
            ops = spec["plus"] + [spec["c"]]
            nplus, has_minus, has_c = len(spec["plus"]), False, True
        w2k = [w2v[pl.ds(k * 16, 16)] for k in range(8)]
        sets = [bscr[0:len(ops)], bscr[5:5 + len(ops)]]
        sems = [semA, semB]

        def issue(c, s):
            for n, (t, col) in enumerate(ops):
                gather(t, col, sets[s][n], c, sems[s])

        def drain(s):
            for n, (t, col) in enumerate(ops):
                pltpu.make_async_copy(
                    tables[t].at[iscr[col - 1].at[pl.ds(0, CH)]],
                    sets[s][n], sems[s]).wait()

        def compute(c, s):
            bset = sets[s]

            def vblock(vb, _):
                def item(j, vec):
                    i = vb * 16 + j
                    acc = None
                    for k in range(8):
                        sl = pl.ds(k * 16, 16)
                        s_ = bset[0][i, sl]
                        for n in range(1, nplus):
                            s_ = s_ + bset[n][i, sl]
                        if has_minus:
                            v = jnp.abs(s_ - bset[nplus][i, sl])
                        elif has_c:
                            v = s_ * bset[nplus][i, sl]
                        else:
                            v = jnp.maximum(s_, 0.0) * w2k[k]
                        acc = v if acc is None else acc + v
                    tot = jnp.sum(acc)
                    return jnp.where(lane == j, tot, vec)

                vec = lax.fori_loop(0, 16, item, jnp.zeros((16,), jnp.float32),
                                    unroll=2)
                lbuf[pl.ds(c * CH + vb * 16, 16)] = -vec if kind == "abs" else vec
                return 0

            lax.fori_loop(0, CH // 16, vblock, 0)

        pipeline(issue, drain, compute)
        pltpu.sync_copy(lbuf, l8_out.at[pl.ds(spec["lrow"] * MPAD + base, PW)])

    def q_task(spec):
        off = spec["qblk"] * MPAD + base
        ta, ca = spec["a"]
        tb, cb = spec["b"]
        sets = [(bscr[0], bscr[1], bscr[2]), (bscr[5], bscr[6], bscr[7])]
        sems = [semA, semB]

        def issue(c, s):
            gather(ta, ca, sets[s][0], c, sems[s])
            gather(tb, cb, sets[s][1], c, sems[s])

        def drain(s):
            for n, (t, col) in enumerate([spec["a"], spec["b"]]):
                pltpu.make_async_copy(
                    tables[t].at[iscr[col - 1].at[pl.ds(0, CH)]],
                    sets[s][n], sems[s]).wait()

        def compute(c, s):
            ba, bb, bq = sets[s]

            def item(i, _):
                for k in range(8):
                    sl = pl.ds(k * 16, 16)
                    bq[i, sl] = ba[i, sl] + bb[i, sl]
                return 0

            lax.fori_loop(0, CH, item, 0, unroll=2)
            pltpu.sync_copy(bq, q_out.at[pl.ds(off + c * CH, CH)])

        pipeline(issue, drain, compute)

    def t_task(spec):
        off = spec["tblk"] * MPAD + base
        tt, ct = spec["t"]
        sets = [bscr[0], bscr[5]]
        sems = [semA, semB]

        def issue(c, s):
            gather(tt, ct, sets[s], c, sems[s])

        def drain(s):
            pltpu.make_async_copy(
                tables[tt].at[iscr[ct - 1].at[pl.ds(0, CH)]],
                sets[s], sems[s]).wait()

        def compute(c, s):
            pltpu.sync_copy(sets[s], t_out.at[pl.ds(off + c * CH, CH)])

        pipeline(issue, drain, compute)

    def task_cols(kind, spec):
        if kind == "relu":
            return [c for _, c in spec["ops"]]
        if kind == "abs":
            return [c for _, c in spec["plus"]] + [spec["minus"][1]]
        if kind == "dot":
            return [c for _, c in spec["plus"]] + [spec["c"][1]]
        if kind == "q":
            return [spec["a"][1], spec["b"][1]]
        return [spec["t"][1]]

    for g in groups:
        gtasks = [(k, s) for k, s in _TASKS if s["g"] == g]
        stage_x(g)
        cols = sorted({c for k, s in gtasks for c in task_cols(k, s)})
        extract_cols(cols)
        for kind, spec in gtasks:
            if kind == "q":
                q_task(spec)
            elif kind == "t":
                t_task(spec)
            else:
                reduce_task(kind, spec)


def _sc_body_qt(e_hbm, r_hbm, xf_hbm, q_out, t_out, *scr):
    _sc_engine([5, 6], [e_hbm, None, r_hbm, None, None], None, xf_hbm,
               None, q_out, t_out, list(scr[0:7]), list(scr[7:17]),
               scr[17], scr[18], scr[19], scr[20], scr[21],
               row_of=lambda g, n: (0 if g == 5 else 410) + n)


def _sc_body_red(e_hbm, c_hbm, r_hbm, a_hbm, b2_hbm, w2_hbm, xf_hbm,
                 l8_out, *scr):
    _sc_engine([0, 1, 2, 3, 4, 7, 8, 9], [e_hbm, c_hbm, r_hbm, a_hbm, b2_hbm],
               w2_hbm, xf_hbm, l8_out, None, None, list(scr[0:7]),
               list(scr[7:17]), scr[17], scr[18], scr[19], scr[20], scr[21])


_SC_SCRATCH = (
    [pltpu.VMEM((PW,), jnp.int32) for _ in range(7)]
    + [pltpu.VMEM((CH, D), jnp.float32) for _ in range(10)]
    + [pltpu.VMEM((13 * 512,), jnp.int32),
       pltpu.VMEM((PW,), jnp.float32),
       pltpu.VMEM((D,), jnp.float32)]
    + [pltpu.SemaphoreType.DMA,
       pltpu.SemaphoreType.DMA]
)


def _sc_call_qt(e_emb, r_emb, xf):
    mesh = plsc.VectorSubcoreMesh(core_axis_name="c", subcore_axis_name="s")
    fn = functools.partial(
        pl.kernel,
        mesh=mesh,
        compiler_params=pltpu.CompilerParams(needs_layout_passes=False),
        out_type=[
            jax.ShapeDtypeStruct((5 * MPAD, D), jnp.float32),
            jax.ShapeDtypeStruct((2 * MPAD, D), jnp.float32),
        ],
        scratch_types=_SC_SCRATCH,
    )(_sc_body_qt)
    return fn(e_emb, r_emb, xf)


def _sc_call_red(e_emb, c_emb, r_emb, A, B2h, w2, xf):
    mesh = plsc.VectorSubcoreMesh(core_axis_name="c", subcore_axis_name="s")
    fn = functools.partial(
        pl.kernel,
        mesh=mesh,
        compiler_params=pltpu.CompilerParams(needs_layout_passes=False),
        out_type=[
            jax.ShapeDtypeStruct((8 * MPAD,), jnp.float32),
        ],
        scratch_types=_SC_SCRATCH,
    )(_sc_body_red)
    return fn(e_emb, c_emb, r_emb, A, B2h, w2, xf)


_BM = 1024
_NB = MPAD // _BM


def _att_body(nq, q_refs, t_ref, w1_ref, b1_ref, w2_ref, out_ref):
    w1 = w1_ref[...].astype(jnp.bfloat16)
    w2 = w2_ref[...].astype(jnp.bfloat16)
    b1 = b1_ref[...]
    qs = [q[...] for q in q_refs]
    qb = [q.astype(jnp.bfloat16) for q in qs]
    zs = [jnp.dot(
        jnp.maximum(jnp.dot(q, w1, preferred_element_type=jnp.float32) + b1,
                    0.0).astype(jnp.bfloat16),
        w2, preferred_element_type=jnp.float32) for q in qb]
    m = zs[0]
    for z in zs[1:]:
        m = jnp.maximum(m, z)
    es = [jnp.exp(z - m) for z in zs]
    den = es[0]
    for e in es[1:]:
        den = den + e
    qe = es[0] * qs[0]
    for e, q in zip(es[1:], qs[1:]):
        qe = qe + e * q
    qe = qe / den
    out_ref[...] = -jnp.sum(jnp.abs(qe - t_ref[...].astype(jnp.float32)), axis=1)


def _att_call(nq, qblks, tblk, qcat, tcat, W1, b1, W2):
    def body(*refs):
        _att_body(nq, refs[0:nq], refs[nq], refs[nq + 1], refs[nq + 2], refs[nq + 3], refs[nq + 4])

    qspecs = [pl.BlockSpec((_BM, D), lambda i, j=jb: (j * _NB + i, 0)) for jb in qblks]
    return pl.pallas_call(
        body,
        grid=(_NB,),
        in_specs=qspecs + [
            pl.BlockSpec((_BM, D), lambda i, j=tblk: (j * _NB + i, 0)),
            pl.BlockSpec((D, D), lambda i: (0, 0)),
            pl.BlockSpec((1, D), lambda i: (0, 0)),
            pl.BlockSpec((D, D), lambda i: (0, 0)),
        ],
        out_specs=pl.BlockSpec((_BM,), lambda i: (i,)),
        out_shape=jax.ShapeDtypeStruct((MPAD,), jnp.float32),
    )(*([qcat] * nq), tcat, W1, b1.reshape(1, D), W2)


def _loss_body(lg_ref, out_ref):
    acc = jnp.zeros((1, D), jnp.float32)
    lane = lax.broadcasted_iota(jnp.int32, (1, D), 1)
    rows = lax.broadcasted_iota(jnp.int32, (MPAD // 64, 63), 0)
    for g in range(10):
        lg = lg_ref[g]
        z = lg[:, 0:1] - lg[:, 1:]
        ls = jnp.minimum(z, 0.0) - jnp.log1p(jnp.exp(-jnp.abs(z)))
        s = jnp.sum(jnp.where(rows < NG[g], ls, 0.0))
        acc = acc + jnp.where(lane == g, -s / (NG[g] * 63), 0.0)
    out_ref[...] = acc


def _loss_call(LG):
    out = pl.pallas_call(
        _loss_body,
        out_shape=jax.ShapeDtypeStruct((1, D), jnp.float32),
    )(LG)
    return out[0, :10]


_QT_ROWS = np.concatenate([np.arange(5, 4096, 10), np.arange(6, 4096, 10)])


def kernel(x, e_emb, c_emb, r_emb, W1, b1, W2, b2, Wcc1, bcc1, Wcc2, bcc2):
    A, B2h = _derive_tables(c_emb, Wcc1, bcc1)
    xf = x.reshape(-1)
    xqt = jnp.take(x, _QT_ROWS, axis=0).reshape(-1)
    qcat, tcat = _sc_call_qt(e_emb, r_emb, xqt)
    (l8,) = _sc_call_red(e_emb, c_emb, r_emb, A, B2h, Wcc2.reshape(D), xf)
    l8 = l8.reshape(8, MPAD)
    l5 = _att_call(2, [0, 1], 0, qcat, tcat, W1, b1, W2)
    l6 = _att_call(3, [2, 3, 4], 1, qcat, tcat, W1, b1, W2)
    LG = jnp.stack([l8[0], l8[1], l8[2], l8[3], l8[4], l5, l6,
                    l8[5], l8[6], l8[7]]).reshape(10, MPAD // 64, 64)
    return _loss_call(LG)


# speedup vs baseline: 4.6495x; 1.0001x over previous
import functools

import jax
import jax.numpy as jnp
import numpy as np
from jax import lax
from jax.experimental import pallas as pl
from jax.experimental.pallas import tpu as pltpu
from jax.experimental.pallas import tpu_sc as plsc

NV = 1000
D = 128
NW = 32
PW = 832
MPAD = NW * PW
CH = 64
NCH = PW // CH
NG = [410] * 6 + [409] * 4

_TASKS = [
    ("relu", dict(g=0, lrow=0, ops=[(3, 6), (4, 7)])),
    ("dot", dict(g=1, lrow=1, plus=[(0, 6)], c=(1, 7))),
    ("abs", dict(g=2, lrow=2, plus=[(0, 5), (2, 6)], minus=(0, 7))),
    ("abs", dict(g=3, lrow=3, plus=[(0, 4), (2, 5), (2, 6)], minus=(0, 7))),
    ("abs", dict(g=4, lrow=4, plus=[(0, 3), (2, 4), (2, 5), (2, 6)], minus=(0, 7))),
    ("q", dict(g=5, qblk=0, a=(0, 3), b=(2, 4))),
    ("q", dict(g=5, qblk=1, a=(0, 5), b=(2, 6))),
    ("t", dict(g=5, tblk=0, t=(0, 7))),
    ("q", dict(g=6, qblk=2, a=(0, 1), b=(2, 2))),
    ("q", dict(g=6, qblk=3, a=(0, 3), b=(2, 4))),
    ("q", dict(g=6, qblk=4, a=(0, 5), b=(2, 6))),
    ("t", dict(g=6, tblk=1, t=(0, 7))),
    ("dot", dict(g=7, lrow=5, plus=[(0, 5), (2, 6)], c=(1, 7))),
    ("dot", dict(g=8, lrow=6, plus=[(0, 4), (2, 5), (2, 6)], c=(1, 7))),
    ("dot", dict(g=9, lrow=7, plus=[(0, 3), (2, 4), (2, 5), (2, 6)], c=(1, 7))),
]


def _k1_body(c_ref, w_ref, b_ref, a_out, b_out):
    cc = c_ref[...]
    a_out[...] = jnp.dot(cc, w_ref[0:D, :], preferred_element_type=jnp.float32)
    b_out[...] = jnp.dot(cc, w_ref[D:2 * D, :], preferred_element_type=jnp.float32) + b_ref[...]


def _derive_tables(c_emb, Wcc1, bcc1):
    return pl.pallas_call(
        _k1_body,
        out_shape=[jax.ShapeDtypeStruct((NV, D), jnp.float32),
                   jax.ShapeDtypeStruct((NV, D), jnp.float32)],
    )(c_emb, Wcc1, bcc1.reshape(1, D))


def _sc_engine(groups, tables, w2_hbm, xf_hbm, l8_out, q_out, t_out,
               iscr, bscr, xrow, lbuf, w2v, semA, semB,
               row_of=lambda g, n: g + 10 * n):
    wid = lax.axis_index("s") * 2 + lax.axis_index("c")
    base = wid * PW
    lane = lax.iota(jnp.int32, 16)

    if w2_hbm is not None:
        pltpu.sync_copy(w2_hbm, w2v)

    def stage_x(g):
        def issue(j, _):
            n_eff = jnp.minimum(wid * 13 + j, NG[g] - 1)
            pltpu.async_copy(
                xf_hbm.at[pl.ds(row_of(g, n_eff) * 512, 512)],
                xrow.at[pl.ds(j * 512, 512)], semA)
            return 0

        def drain(j, _):
            pltpu.make_async_copy(
                xf_hbm.at[pl.ds(0, 512)], xrow.at[pl.ds(0, 512)], semA).wait()
            return 0

        lax.fori_loop(0, 13, issue, 0)
        lax.fori_loop(0, 13, drain, 0)

    def extract_cols(cols):
        def ext(vb, _):
            p = vb * 16 + lane
            fb = (p >> 6) * 512 + (p & 63) * 8
            for col in cols:
                v = plsc.load_gather(xrow, [fb + col])
                iscr[col - 1][pl.ds(vb * 16, 16)] = v
            return 0

        lax.fori_loop(0, PW // 16, ext, 0, unroll=2)

    def gather(tab_slot, col, buf, c, sem):
        return pltpu.async_copy(
            tables[tab_slot].at[iscr[col - 1].at[pl.ds(c * CH, CH)]],
            buf, sem)

    def pipeline(issue, drain, compute):
        issue(jnp.int32(0), 0)

        def pair(i, _):
            cA = 2 * i
            issue(cA + 1, 1)
            drain(0)
            compute(cA, 0)
            issue(cA + 2, 0)
            drain(1)
            compute(cA + 1, 1)
            return 0

        lax.fori_loop(0, (NCH - 1) // 2, pair, 0)
        drain(0)
        compute(jnp.int32(NCH - 1), 0)

    def reduce_task(kind, spec):
        if kind == "relu":
            ops = spec["ops"]
            nplus, has_minus, has_c = len(ops), False, False
        elif kind == "abs":
            ops = spec["plus"] + [spec["minus"]]
            nplus, has_minus, has_c = len(spec["plus"]), True, False
        else:
            ops = spec["plus"] + [spec["c"]]
            nplus, has_minus, has_c = len(spec["plus"]), False, True
        w2k = [w2v[pl.ds(k * 16, 16)] for k in range(8)]
        sets = [bscr[0:len(ops)], bscr[5:5 + len(ops)]]
        sems = [semA, semB]

        def issue(c, s):
            for n, (t, col) in enumerate(ops):
                gather(t, col, sets[s][n], c, sems[s])

        def drain(s):
            for n, (t, col) in enumerate(ops):
                pltpu.make_async_copy(
                    tables[t].at[iscr[col - 1].at[pl.ds(0, CH)]],
                    sets[s][n], sems[s]).wait()

        def compute(c, s):
            bset = sets[s]

            def vblock(vb, _):
                def item(j, vec):
                    i = vb * 16 + j
                    acc = None
                    for k in range(8):
                        sl = pl.ds(k * 16, 16)
                        s_ = bset[0][i, sl]
                        for n in range(1, nplus):
                            s_ = s_ + bset[n][i, sl]
                        if has_minus:
                            v = jnp.abs(s_ - bset[nplus][i, sl])
                        elif has_c:
                            v = s_ * bset[nplus][i, sl]
                        else:
                            v = jnp.maximum(s_, 0.0) * w2k[k]
                        acc = v if acc is None else acc + v
                    tot = jnp.sum(acc)
                    return jnp.where(lane == j, tot, vec)

                vec = lax.fori_loop(0, 16, item, jnp.zeros((16,), jnp.float32),
                                    unroll=2)
                lbuf[pl.ds(c * CH + vb * 16, 16)] = -vec if kind == "abs" else vec
                return 0

            lax.fori_loop(0, CH // 16, vblock, 0)

        pipeline(issue, drain, compute)
        pltpu.sync_copy(lbuf, l8_out.at[pl.ds(spec["lrow"] * MPAD + base, PW)])

    def q_task(spec):
        off = spec["qblk"] * MPAD + base
        ta, ca = spec["a"]
        tb, cb = spec["b"]
        sets = [(bscr[0], bscr[1], bscr[2]), (bscr[5], bscr[6], bscr[7])]
        sems = [semA, semB]

        def issue(c, s):
            gather(ta, ca, sets[s][0], c, sems[s])
            gather(tb, cb, sets[s][1], c, sems[s])

        def drain(s):
            for n, (t, col) in enumerate([spec["a"], spec["b"]]):
                pltpu.make_async_copy(
                    tables[t].at[iscr[col - 1].at[pl.ds(0, CH)]],
                    sets[s][n], sems[s]).wait()

        def compute(c, s):
            ba, bb, bq = sets[s]

            def item(i, _):
                for k in range(8):
                    sl = pl.ds(k * 16, 16)
                    bq[i, sl] = ba[i, sl] + bb[i, sl]
                return 0

            lax.fori_loop(0, CH, item, 0, unroll=2)
            pltpu.sync_copy(bq, q_out.at[pl.ds(off + c * CH, CH)])

        pipeline(issue, drain, compute)

    def t_task(spec):
        off = spec["tblk"] * MPAD + base
        tt, ct = spec["t"]
        sets = [bscr[0], bscr[5]]
        sems = [semA, semB]

        def issue(c, s):
            gather(tt, ct, sets[s], c, sems[s])

        def drain(s):
            pltpu.make_async_copy(
                tables[tt].at[iscr[ct - 1].at[pl.ds(0, CH)]],
                sets[s], sems[s]).wait()

        def compute(c, s):
            pltpu.sync_copy(sets[s], t_out.at[pl.ds(off + c * CH, CH)])

        pipeline(issue, drain, compute)

    def task_cols(kind, spec):
        if kind == "relu":
            return [c for _, c in spec["ops"]]
        if kind == "abs":
            return [c for _, c in spec["plus"]] + [spec["minus"][1]]
        if kind == "dot":
            return [c for _, c in spec["plus"]] + [spec["c"][1]]
        if kind == "q":
            return [spec["a"][1], spec["b"][1]]
        return [spec["t"][1]]

    for g in groups:
        gtasks = [(k, s) for k, s in _TASKS if s["g"] == g]
        stage_x(g)
        cols = sorted({c for k, s in gtasks for c in task_cols(k, s)})
        extract_cols(cols)
        for kind, spec in gtasks:
            if kind == "q":
                q_task(spec)
            elif kind == "t":
                t_task(spec)
            else:
                reduce_task(kind, spec)


def _sc_body_qt(e_hbm, r_hbm, xf_hbm, q_out, t_out, *scr):
    _sc_engine([5, 6], [e_hbm, None, r_hbm, None, None], None, xf_hbm,
               None, q_out, t_out, list(scr[0:7]), list(scr[7:17]),
               scr[17], scr[18], scr[19], scr[20], scr[21],
               row_of=lambda g, n: (0 if g == 5 else 410) + n)


def _sc_body_red(e_hbm, c_hbm, r_hbm, a_hbm, b2_hbm, w2_hbm, xf_hbm,
                 l8_out, *scr):
    _sc_engine([0, 1, 2, 3, 4, 7, 8, 9], [e_hbm, c_hbm, r_hbm, a_hbm, b2_hbm],
               w2_hbm, xf_hbm, l8_out, None, None, list(scr[0:7]),
               list(scr[7:17]), scr[17], scr[18], scr[19], scr[20], scr[21])


_SC_SCRATCH = (
    [pltpu.VMEM((PW,), jnp.int32) for _ in range(7)]
    + [pltpu.VMEM((CH, D), jnp.float32) for _ in range(10)]
    + [pltpu.VMEM((13 * 512,), jnp.int32),
       pltpu.VMEM((PW,), jnp.float32),
       pltpu.VMEM((D,), jnp.float32)]
    + [pltpu.SemaphoreType.DMA,
       pltpu.SemaphoreType.DMA]
)


def _sc_call_qt(e_emb, r_emb, xf):
    mesh = plsc.VectorSubcoreMesh(core_axis_name="c", subcore_axis_name="s")
    fn = functools.partial(
        pl.kernel,
        mesh=mesh,
        compiler_params=pltpu.CompilerParams(needs_layout_passes=False),
        out_type=[
            jax.ShapeDtypeStruct((5 * MPAD, D), jnp.float32),
            jax.ShapeDtypeStruct((2 * MPAD, D), jnp.float32),
        ],
        scratch_types=_SC_SCRATCH,
    )(_sc_body_qt)
    return fn(e_emb, r_emb, xf)


def _sc_call_red(e_emb, c_emb, r_emb, A, B2h, w2, xf):
    mesh = plsc.VectorSubcoreMesh(core_axis_name="c", subcore_axis_name="s")
    fn = functools.partial(
        pl.kernel,
        mesh=mesh,
        compiler_params=pltpu.CompilerParams(needs_layout_passes=False),
        out_type=[
            jax.ShapeDtypeStruct((8 * MPAD,), jnp.float32),
        ],
        scratch_types=_SC_SCRATCH,
    )(_sc_body_red)
    return fn(e_emb, c_emb, r_emb, A, B2h, w2, xf)


_BM = 1024
_NB = MPAD // _BM


def _att_body(nq, q_refs, t_ref, w1_ref, b1_ref, w2_ref, out_ref):
    w1 = w1_ref[...].astype(jnp.bfloat16)
    w2 = w2_ref[...].astype(jnp.bfloat16)
    b1 = b1_ref[...]
    qs = [q[...] for q in q_refs]
    qb = [q.astype(jnp.bfloat16) for q in qs]
    zs = [jnp.dot(
        jnp.maximum(jnp.dot(q, w1, preferred_element_type=jnp.float32) + b1,
                    0.0).astype(jnp.bfloat16),
        w2, preferred_element_type=jnp.float32) for q in qb]
    m = zs[0]
    for z in zs[1:]:
        m = jnp.maximum(m, z)
    es = [jnp.exp(z - m) for z in zs]
    den = es[0]
    for e in es[1:]:
        den = den + e
    qe = es[0] * qs[0]
    for e, q in zip(es[1:], qs[1:]):
        qe = qe + e * q
    qe = qe / den
    out_ref[...] = -jnp.sum(jnp.abs(qe - t_ref[...].astype(jnp.float32)), axis=1)


def _att_call(nq, qblks, tblk, qcat, tcat, W1, b1, W2):
    def body(*refs):
        _att_body(nq, refs[0:nq], refs[nq], refs[nq + 1], refs[nq + 2], refs[nq + 3], refs[nq + 4])

    qspecs = [pl.BlockSpec((_BM, D), lambda i, j=jb: (j * _NB + i, 0)) for jb in qblks]
    return pl.pallas_call(
        body,
        grid=(_NB,),
        in_specs=qspecs + [
            pl.BlockSpec((_BM, D), lambda i, j=tblk: (j * _NB + i, 0)),
            pl.BlockSpec((D, D), lambda i: (0, 0)),
            pl.BlockSpec((1, D), lambda i: (0, 0)),
            pl.BlockSpec((D, D), lambda i: (0, 0)),
        ],
        out_specs=pl.BlockSpec((_BM,), lambda i: (i,)),
        out_shape=jax.ShapeDtypeStruct((MPAD,), jnp.float32),
    )(*([qcat] * nq), tcat, W1, b1.reshape(1, D), W2)


def _loss_body(lg_ref, out_ref):
    acc = jnp.zeros((1, D), jnp.float32)
    lane = lax.broadcasted_iota(jnp.int32, (1, D), 1)
    rows = lax.broadcasted_iota(jnp.int32, (MPAD // 64, 63), 0)
    for g in range(10):
        lg = lg_ref[g]
        z = lg[:, 0:1] - lg[:, 1:]
        ls = jnp.minimum(z, 0.0) - jnp.log1p(jnp.exp(-jnp.abs(z)))
        s = jnp.sum(jnp.where(rows < NG[g], ls, 0.0))
        acc = acc + jnp.where(lane == g, -s / (NG[g] * 63), 0.0)
    out_ref[...] = acc


def _loss_call(LG):
    out = pl.pallas_call(
        _loss_body,
        out_shape=jax.ShapeDtypeStruct((1, D), jnp.float32),
    )(LG)
    return out[0, :10]


_QT_ROWS = np.concatenate([np.arange(5, 4096, 10), np.arange(6, 4096, 10)])


def kernel(x, e_emb, c_emb, r_emb, W1, b1, W2, b2, Wcc1, bcc1, Wcc2, bcc2):
    A, B2h = _derive_tables(c_emb, Wcc1, bcc1)
    xf = x.reshape(-1)
    xqt = jnp.take(x, _QT_ROWS, axis=0).reshape(-1)
    qcat, tcat = _sc_call_qt(e_emb, r_emb, xqt)
    (l8,) = _sc_call_red(e_emb, c_emb, r_emb, A, B2h, Wcc2.reshape(D), xf)
    l8 = l8.reshape(8, MPAD)
    l5 = _att_call(2, [0, 1], 0, qcat, tcat, W1, b1, W2)
    l6 = _att_call(3, [2, 3, 4], 1, qcat, tcat, W1, b1, W2)
    LG = jnp.stack([l8[0], l8[1], l8[2], l8[3], l8[4], l5, l6,
                    l8[5], l8[6], l8[7]]).reshape(10, MPAD // 64, 64)
    return _loss_call(LG)
